# baseline (speedup 1.0000x reference)
import functools

import jax
import jax.numpy as jnp
import numpy as np
from jax import lax
from jax.experimental import pallas as pl
from jax.experimental.pallas import tpu as pltpu
from jax.experimental.pallas import tpu_sc as plsc

NU = 25000
NI = 25000
N = NU + NI
K = 32
EMB = 128
E = 1600000
B = 16384
WD = 1e-4

NC, NS = 2, 16
NW = NC * NS
ZR = 25088
STRIPE = ZR // NS
ZCH = STRIPE // 4
EP = 1638400
CR = EP // 128
CPW = CR // NW
GRP = CPW // 16
BCR = B // 128
BPW = BCR // NW

_MESH = plsc.VectorSubcoreMesh(core_axis_name="c", subcore_axis_name="s",
                               num_cores=NC, num_subcores=NS)


def _sc_a_body(rows2d, cols2d, theta, uemb, iemb, users2d, pos2d, neg2d,
               zeros_z, z_out, u_out, p_out, n_out, thu_out,
               rbuf, cbuf, g0, g1, gemb, eidx, zbuf, z_sh, sem0, sem1, sem2):
    c = lax.axis_index("c")
    s = lax.axis_index("s")
    wid = c * NS + s

    pltpu.sync_copy(zeros_z, zbuf)

    if True:
        for q in range(4):
            pltpu.sync_copy(zbuf, z_sh.at[pl.ds(s * STRIPE + q * ZCH, ZCH)])
        plsc.subcore_barrier()

        def grp(g, carry):
            base = wid * CPW + g * 16
            pltpu.sync_copy(rows2d.at[pl.ds(base, 16)], rbuf)
            pltpu.sync_copy(cols2d.at[pl.ds(base, 16)], cbuf)
            for i in range(16):
                for l in range(8):
                    sl = pl.ds(l * 16, 16)
                    r = rbuf[i, sl]
                    cc = cbuf[i, sl]
                    keep = (r >= NU) & (cc < NU)
                    rbuf[i, sl] = jnp.where(keep, r - NU, NI)
                    cbuf[i, sl] = jnp.where(keep, cc, 0)
            cp = pltpu.async_copy(theta.at[cbuf.at[0]], g0, sem0)
            for j in range(16):
                cur, nxt = (g0, g1) if j % 2 == 0 else (g1, g0)
                nsem = sem1 if j % 2 == 0 else sem0
                cp.wait()
                if j < 15:
                    cp = pltpu.async_copy(theta.at[cbuf.at[j + 1]], nxt,
                                          nsem)
                pltpu.sync_copy(cur, z_sh.at[rbuf.at[j]], add=True)
            return carry

        lax.fori_loop(0, GRP, grp, 0)

        for tbl, idx2d, out, buf in ((uemb, users2d, u_out, gemb),
                                     (iemb, pos2d, p_out, gemb),
                                     (iemb, neg2d, n_out, gemb),
                                     (theta, users2d, thu_out, g0)):
            pltpu.sync_copy(idx2d.at[pl.ds((wid // 2) * 8, 8)], eidx)
            for j in range(BPW):
                jj = (wid % 2) * BPW + j
                pltpu.async_copy(tbl.at[eidx.at[jj]], buf, sem2).wait()
                pltpu.sync_copy(buf,
                                out.at[pl.ds((wid * BPW + j) * 128, 128)])

        plsc.subcore_barrier()
        for q in range(4):
            r0 = s * STRIPE + q * ZCH
            pltpu.sync_copy(z_sh.at[pl.ds(r0, ZCH)], zbuf)
            pltpu.sync_copy(zbuf, z_out.at[c, pl.ds(r0, ZCH)])


_sc_a = pl.kernel(
    _sc_a_body,
    out_type=(jax.ShapeDtypeStruct((NC, ZR, K), jnp.float32),
              jax.ShapeDtypeStruct((B, EMB), jnp.float32),
              jax.ShapeDtypeStruct((B, EMB), jnp.float32),
              jax.ShapeDtypeStruct((B, EMB), jnp.float32),
              jax.ShapeDtypeStruct((B, K), jnp.float32)),
    mesh=_MESH,
    scratch_types=[
        pltpu.VMEM((16, 128), jnp.int32),
        pltpu.VMEM((16, 128), jnp.int32),
        pltpu.VMEM((128, K), jnp.float32),
        pltpu.VMEM((128, K), jnp.float32),
        pltpu.VMEM((128, EMB), jnp.float32),
        pltpu.VMEM((8, 128), jnp.int32),
        pltpu.VMEM((ZCH, K), jnp.float32),
        pltpu.VMEM_SHARED((ZR, K), jnp.float32),
        pltpu.SemaphoreType.DMA,
        pltpu.SemaphoreType.DMA,
        pltpu.SemaphoreType.DMA,
    ],
    compiler_params=pltpu.CompilerParams(use_tc_tiling_on_sc=False),
)


def _sc_b_body(z1, pos2d, neg2d, z1p_out, z1n_out, eidx, gbuf, sem):
    c = lax.axis_index("c")
    s = lax.axis_index("s")
    wid = c * NS + s
    for idx2d, out in ((pos2d, z1p_out), (neg2d, z1n_out)):
        pltpu.sync_copy(idx2d.at[pl.ds((wid // 2) * 8, 8)], eidx)
        for j in range(BPW):
            jj = (wid % 2) * BPW + j
            pltpu.async_copy(z1.at[eidx.at[jj]], gbuf, sem).wait()
            pltpu.sync_copy(gbuf, out.at[pl.ds((wid * BPW + j) * 128, 128)])


_sc_b = pl.kernel(
    _sc_b_body,
    out_type=(jax.ShapeDtypeStruct((B, K), jnp.float32),
              jax.ShapeDtypeStruct((B, K), jnp.float32)),
    mesh=_MESH,
    scratch_types=[
        pltpu.VMEM((8, 128), jnp.int32),
        pltpu.VMEM((128, K), jnp.float32),
        pltpu.SemaphoreType.DMA,
    ],
    compiler_params=pltpu.CompilerParams(use_tc_tiling_on_sc=False),
)


def _tc1_body(tu_ref, w1_ref, w2_ref, th_ref, s3_ref, s2_ref):
    x = tu_ref[...]
    m = jnp.max(x, axis=1, keepdims=True)
    e = jnp.exp(x - m)
    th_ref[...] = e / jnp.sum(e, axis=1, keepdims=True)
    s3_ref[...] = jnp.sum(x * x).reshape(1, 1)
    w1 = w1_ref[...]
    w2 = w2_ref[...]
    s2_ref[...] = (jnp.sum(w1 * w1) + jnp.sum(w2 * w2)).reshape(1, 1)


_tc1 = pl.pallas_call(
    _tc1_body,
    out_shape=(jax.ShapeDtypeStruct((NU, K), jnp.float32),
               jax.ShapeDtypeStruct((1, 1), jnp.float32),
               jax.ShapeDtypeStruct((1, 1), jnp.float32)),
)


def _tc2a_body(z0_ref, z1_ref, w1_ref, w2_ref, sc_ref, out_ref):
    z = (z0_ref[...] + z1_ref[...]) * sc_ref[0, 0]
    t = z * w1_ref[...] + w2_ref[...]
    out_ref[...] = 1.0 / (1.0 + jnp.exp(-t))


_tc2a = pl.pallas_call(
    _tc2a_body,
    grid=(5,),
    in_specs=[pl.BlockSpec((5000, K), lambda i: (i, 0)),
              pl.BlockSpec((5000, K), lambda i: (i, 0)),
              pl.BlockSpec((5000, 1), lambda i: (i, 0)),
              pl.BlockSpec((5000, 1), lambda i: (i, 0)),
              pl.BlockSpec((1, 1), lambda i: (0, 0))],
    out_specs=pl.BlockSpec((5000, K), lambda i: (i, 0)),
    out_shape=jax.ShapeDtypeStruct((NI, K), jnp.float32),
)


def _tc2b_body(u_ref, p_ref, n_ref, ps_ref, ns_ref, sq_ref):
    i = pl.program_id(0)
    u = u_ref[...]
    p = p_ref[...]
    n = n_ref[...]
    ps_ref[...] = jnp.sum(u * p, axis=1, keepdims=True)
    ns_ref[...] = jnp.sum(u * n, axis=1, keepdims=True)
    acc = jnp.sum(u * u) + jnp.sum(p * p) + jnp.sum(n * n)

    @pl.when(i == 0)
    def _():
        sq_ref[...] = acc.reshape(1, 1)

    @pl.when(i > 0)
    def _():
        sq_ref[...] += acc.reshape(1, 1)


_tc2b = pl.pallas_call(
    _tc2b_body,
    grid=(4,),
    in_specs=[pl.BlockSpec((4096, EMB), lambda i: (i, 0))] * 3,
    out_specs=(pl.BlockSpec((4096, 1), lambda i: (i, 0)),
               pl.BlockSpec((4096, 1), lambda i: (i, 0)),
               pl.BlockSpec((1, 1), lambda i: (0, 0))),
    out_shape=(jax.ShapeDtypeStruct((B, 1), jnp.float32),
               jax.ShapeDtypeStruct((B, 1), jnp.float32),
               jax.ShapeDtypeStruct((1, 1), jnp.float32)),
)


def _tc3_body(ps_ref, ns_ref, thu_ref, z1p_ref, z1n_ref, s2_ref, s3_ref,
              sq_ref, out_ref, acc):
    i = pl.program_id(0)
    thu = thu_ref[...]
    gp = jnp.sum(thu * z1p_ref[...], axis=1, keepdims=True)
    gn = jnp.sum(thu * z1n_ref[...], axis=1, keepdims=True)
    rp = 1.0 / (1.0 + jnp.exp(-ps_ref[...]))
    rn = 1.0 / (1.0 + jnp.exp(-ns_ref[...]))
    mf = jnp.sum(gp * -jnp.log(rp)) + jnp.sum(gn * -jnp.log(1.0 - rn))
    l1 = -float(np.log(np.float32(0.001)))
    l0 = -float(np.log(np.float32(1.0) - np.float32(0.001)))
    unk = l1 * jnp.sum(1.0 - gp) + l0 * jnp.sum(1.0 - gn)
    gu = -(jnp.sum(gp * jnp.log(gp) + (1.0 - gp) * jnp.log(1.0 - gp))
           + jnp.sum(gn * jnp.log(gn) + (1.0 - gn) * jnp.log(1.0 - gn)))

    @pl.when(i == 0)
    def _():
        acc[0] = mf
        acc[1] = unk
        acc[2] = gu

    @pl.when(i > 0)
    def _():
        acc[0] += mf
        acc[1] += unk
        acc[2] += gu

    @pl.when(i == 3)
    def _():
        rl1 = 0.5 * sq_ref[0, 0] / float(B)
        rl2 = 0.5 * s2_ref[0, 0] / float(NI)
        rl3 = 0.5 * s3_ref[0, 0] / float(NU)
        reg = WD * (rl1 + rl3) + 0.1 * rl2
        inv = 1.0 / float(2 * B)
        out_ref[...] = (acc[0] * inv + 0.1 * (acc[1] * inv - acc[2] * inv)
                        + reg).reshape(1, 1)


_tc3 = pl.pallas_call(
    _tc3_body,
    grid=(4,),
    in_specs=[pl.BlockSpec((4096, 1), lambda i: (i, 0)),
              pl.BlockSpec((4096, 1), lambda i: (i, 0)),
              pl.BlockSpec((4096, K), lambda i: (i, 0)),
              pl.BlockSpec((4096, K), lambda i: (i, 0)),
              pl.BlockSpec((4096, K), lambda i: (i, 0)),
              pl.BlockSpec((1, 1), lambda i: (0, 0)),
              pl.BlockSpec((1, 1), lambda i: (0, 0)),
              pl.BlockSpec((1, 1), lambda i: (0, 0))],
    out_specs=pl.BlockSpec((1, 1), lambda i: (0, 0)),
    out_shape=jax.ShapeDtypeStruct((1, 1), jnp.float32),
    scratch_shapes=[pltpu.SMEM((4,), jnp.float32)],
)


def kernel(users, positive_items, negative_items, edge_index, edge_values,
           user_embedding, item_embedding, theta_user, w1, w2):
    users = users.astype(jnp.int32)
    positive_items = positive_items.astype(jnp.int32)
    negative_items = negative_items.astype(jnp.int32)
    edge_index = edge_index.astype(jnp.int32)

    theta, s3, s2 = _tc1(theta_user, w1.reshape(8, NU // 8),
                         w2.reshape(8, NI // 8))

    pad = EP - E
    rows2d = jnp.concatenate(
        [edge_index[0], jnp.zeros((pad,), dtype=jnp.int32)]).reshape(CR, 128)
    cols2d = jnp.concatenate(
        [edge_index[1], jnp.zeros((pad,), dtype=jnp.int32)]).reshape(CR, 128)
    zeros_z = jnp.zeros((ZCH, K), jnp.float32)
    users2d = users.reshape(BCR, 128)
    pos2d = positive_items.reshape(BCR, 128)
    neg2d = negative_items.reshape(BCR, 128)

    z_parts, u, p, n, thu = _sc_a(rows2d, cols2d, theta, user_embedding,
                                  item_embedding, users2d, pos2d, neg2d,
                                  zeros_z)

    scale = edge_values[0].reshape(1, 1)
    z1 = _tc2a(z_parts[0, :NI], z_parts[1, :NI], w1, w2, scale)
    ps, ns, sq = _tc2b(u, p, n)
    z1p, z1n = _sc_b(z1, pos2d, neg2d)
    loss = _tc3(ps, ns, thu, z1p, z1n, s2, s3, sq)
    return loss.reshape(())


# speedup vs baseline: 1.0016x; 1.0016x over previous
import functools

import jax
import jax.numpy as jnp
import numpy as np
from jax import lax
from jax.experimental import pallas as pl
from jax.experimental.pallas import tpu as pltpu
from jax.experimental.pallas import tpu_sc as plsc

NU = 25000
NI = 25000
N = NU + NI
K = 32
EMB = 128
E = 1600000
B = 16384
WD = 1e-4

NC, NS = 2, 16
NW = NC * NS
ZR = 25088
STRIPE = ZR // NS
ZCH = STRIPE // 28
EP = 1638400
CR = EP // 128
CPW = CR // NW
GRP = CPW // 16
BCR = B // 128
BPW = BCR // NW

_MESH = plsc.VectorSubcoreMesh(core_axis_name="c", subcore_axis_name="s",
                               num_cores=NC, num_subcores=NS)


def _sc_a_body(rows2d, cols2d, theta, uemb, iemb, users2d, pos2d, neg2d,
               zeros_z, z_out, u_out, p_out, n_out, thu_out,
               rbuf, cbuf, g0, gemb, gAB, eidx, zbuf, z_sh,
               sem0, sem1, sem2, sem3):
    c = lax.axis_index("c")
    s = lax.axis_index("s")
    wid = c * NS + s

    pltpu.sync_copy(zeros_z, zbuf)

    if True:
        for q in range(28):
            pltpu.sync_copy(zbuf, z_sh.at[pl.ds(s * STRIPE + q * ZCH, ZCH)])
        plsc.subcore_barrier()

        def grp(g, carry):
            base = wid * CPW + g * 16
            pltpu.sync_copy(rows2d.at[pl.ds(base, 16)], rbuf)
            pltpu.sync_copy(cols2d.at[pl.ds(base, 16)], cbuf)
            for i in range(16):
                for l in range(8):
                    sl = pl.ds(l * 16, 16)
                    r = rbuf[i, sl]
                    cc = cbuf[i, sl]
                    keep = (r >= NU) & (cc < NU)
                    rbuf[i, sl] = jnp.where(keep, r - NU, NI)
                    cbuf[i, sl] = jnp.where(keep, cc, 0)
            gsems = (sem0, sem1)
            ssems = (sem2, sem3)
            gaths = [None] * 4
            scats = [None] * 4
            for w in range(4):
                h = w % 2
                if w >= 2:
                    for cp in scats[w - 2]:
                        cp.wait()
                gaths[w] = [pltpu.async_copy(
                    theta.at[cbuf.at[4 * w + j]],
                    gAB.at[pl.ds((4 * h + j) * 128, 128)], gsems[h])
                    for j in range(4)]
                if w >= 1:
                    for cp in gaths[w - 1]:
                        cp.wait()
                    hp = (w - 1) % 2
                    scats[w - 1] = [pltpu.async_copy(
                        gAB.at[pl.ds((4 * hp + j) * 128, 128)],
                        z_sh.at[rbuf.at[4 * (w - 1) + j]], ssems[hp],
                        add=True) for j in range(4)]
            for cp in gaths[3]:
                cp.wait()
            scats[3] = [pltpu.async_copy(
                gAB.at[pl.ds((4 + j) * 128, 128)],
                z_sh.at[rbuf.at[12 + j]], ssems[1], add=True)
                for j in range(4)]
            for cp in scats[2] + scats[3]:
                cp.wait()
            return carry

        lax.fori_loop(0, GRP, grp, 0)

        wb = {}
        for tbl, idx2d, out, keys in ((uemb, users2d, u_out, ("a", "b")),
                                      (iemb, pos2d, p_out, ("a", "b")),
                                      (iemb, neg2d, n_out, ("a", "b")),
                                      (theta, users2d, thu_out, ("t", "t"))):
            slots = {"a": gemb.at[pl.ds(0, 128)],
                     "b": gemb.at[pl.ds(128, 128)], "t": g0}
            pltpu.sync_copy(idx2d.at[pl.ds((wid // 2) * 8, 8)], eidx)
            for j in range(BPW):
                jj = (wid % 2) * BPW + j
                k = keys[j % 2]
                buf = slots[k]
                if wb.get(k) is not None:
                    wb[k].wait()
                    wb[k] = None
                pltpu.async_copy(tbl.at[eidx.at[jj]], buf, sem0).wait()
                wb[k] = pltpu.async_copy(
                    buf, out.at[pl.ds((wid * BPW + j) * 128, 128)], sem2)
        for k in list(wb):
            if wb[k] is not None:
                wb[k].wait()

        plsc.subcore_barrier()
        for q in range(7):
            r0 = s * STRIPE + q * ZCH
            pltpu.sync_copy(z_sh.at[pl.ds(r0, ZCH)], zbuf)
            pltpu.sync_copy(zbuf, z_out.at[c, pl.ds(r0, ZCH)])


_sc_a = pl.kernel(
    _sc_a_body,
    out_type=(jax.ShapeDtypeStruct((NC, ZR, K), jnp.float32),
              jax.ShapeDtypeStruct((B, EMB), jnp.float32),
              jax.ShapeDtypeStruct((B, EMB), jnp.float32),
              jax.ShapeDtypeStruct((B, EMB), jnp.float32),
              jax.ShapeDtypeStruct((B, K), jnp.float32)),
    mesh=_MESH,
    scratch_types=[
        pltpu.VMEM((16, 128), jnp.int32),
        pltpu.VMEM((16, 128), jnp.int32),
        pltpu.VMEM((128, K), jnp.float32),
        pltpu.VMEM((256, EMB), jnp.float32),
        pltpu.VMEM((1024, K), jnp.float32),
        pltpu.VMEM((8, 128), jnp.int32),
        pltpu.VMEM((ZCH, K), jnp.float32),
        pltpu.VMEM_SHARED((ZR, K), jnp.float32),
        pltpu.SemaphoreType.DMA,
        pltpu.SemaphoreType.DMA,
        pltpu.SemaphoreType.DMA,
        pltpu.SemaphoreType.DMA,
    ],
    compiler_params=pltpu.CompilerParams(use_tc_tiling_on_sc=False),
)


def _sc_b_body(z1, pos2d, neg2d, z1p_out, z1n_out, eidx, gbuf, sem):
    c = lax.axis_index("c")
    s = lax.axis_index("s")
    wid = c * NS + s
    for idx2d, out in ((pos2d, z1p_out), (neg2d, z1n_out)):
        pltpu.sync_copy(idx2d.at[pl.ds((wid // 2) * 8, 8)], eidx)
        for j in range(BPW):
            jj = (wid % 2) * BPW + j
            pltpu.async_copy(z1.at[eidx.at[jj]], gbuf, sem).wait()
            pltpu.sync_copy(gbuf, out.at[pl.ds((wid * BPW + j) * 128, 128)])


_sc_b = pl.kernel(
    _sc_b_body,
    out_type=(jax.ShapeDtypeStruct((B, K), jnp.float32),
              jax.ShapeDtypeStruct((B, K), jnp.float32)),
    mesh=_MESH,
    scratch_types=[
        pltpu.VMEM((8, 128), jnp.int32),
        pltpu.VMEM((128, K), jnp.float32),
        pltpu.SemaphoreType.DMA,
    ],
    compiler_params=pltpu.CompilerParams(use_tc_tiling_on_sc=False),
)


def _tc1_body(tu_ref, w1_ref, w2_ref, th_ref, s3_ref, s2_ref):
    x = tu_ref[...]
    m = jnp.max(x, axis=1, keepdims=True)
    e = jnp.exp(x - m)
    th_ref[...] = e / jnp.sum(e, axis=1, keepdims=True)
    s3_ref[...] = jnp.sum(x * x).reshape(1, 1)
    w1 = w1_ref[...]
    w2 = w2_ref[...]
    s2_ref[...] = (jnp.sum(w1 * w1) + jnp.sum(w2 * w2)).reshape(1, 1)


_tc1 = pl.pallas_call(
    _tc1_body,
    out_shape=(jax.ShapeDtypeStruct((NU, K), jnp.float32),
               jax.ShapeDtypeStruct((1, 1), jnp.float32),
               jax.ShapeDtypeStruct((1, 1), jnp.float32)),
)


def _tc2a_body(z0_ref, z1_ref, w1_ref, w2_ref, sc_ref, out_ref):
    z = (z0_ref[...] + z1_ref[...]) * sc_ref[0, 0]
    t = z * w1_ref[...] + w2_ref[...]
    out_ref[...] = 1.0 / (1.0 + jnp.exp(-t))


_tc2a = pl.pallas_call(
    _tc2a_body,
    grid=(5,),
    in_specs=[pl.BlockSpec((5000, K), lambda i: (i, 0)),
              pl.BlockSpec((5000, K), lambda i: (i, 0)),
              pl.BlockSpec((5000, 1), lambda i: (i, 0)),
              pl.BlockSpec((5000, 1), lambda i: (i, 0)),
              pl.BlockSpec((1, 1), lambda i: (0, 0))],
    out_specs=pl.BlockSpec((5000, K), lambda i: (i, 0)),
    out_shape=jax.ShapeDtypeStruct((NI, K), jnp.float32),
)


def _tc2b_body(u_ref, p_ref, n_ref, ps_ref, ns_ref, sq_ref):
    i = pl.program_id(0)
    u = u_ref[...]
    p = p_ref[...]
    n = n_ref[...]
    ps_ref[...] = jnp.sum(u * p, axis=1, keepdims=True)
    ns_ref[...] = jnp.sum(u * n, axis=1, keepdims=True)
    acc = jnp.sum(u * u) + jnp.sum(p * p) + jnp.sum(n * n)

    @pl.when(i == 0)
    def _():
        sq_ref[...] = acc.reshape(1, 1)

    @pl.when(i > 0)
    def _():
        sq_ref[...] += acc.reshape(1, 1)


_tc2b = pl.pallas_call(
    _tc2b_body,
    grid=(4,),
    in_specs=[pl.BlockSpec((4096, EMB), lambda i: (i, 0))] * 3,
    out_specs=(pl.BlockSpec((4096, 1), lambda i: (i, 0)),
               pl.BlockSpec((4096, 1), lambda i: (i, 0)),
               pl.BlockSpec((1, 1), lambda i: (0, 0))),
    out_shape=(jax.ShapeDtypeStruct((B, 1), jnp.float32),
               jax.ShapeDtypeStruct((B, 1), jnp.float32),
               jax.ShapeDtypeStruct((1, 1), jnp.float32)),
)


def _tc3_body(ps_ref, ns_ref, thu_ref, z1p_ref, z1n_ref, s2_ref, s3_ref,
              sq_ref, out_ref, acc):
    i = pl.program_id(0)
    thu = thu_ref[...]
    gp = jnp.sum(thu * z1p_ref[...], axis=1, keepdims=True)
    gn = jnp.sum(thu * z1n_ref[...], axis=1, keepdims=True)
    rp = 1.0 / (1.0 + jnp.exp(-ps_ref[...]))
    rn = 1.0 / (1.0 + jnp.exp(-ns_ref[...]))
    mf = jnp.sum(gp * -jnp.log(rp)) + jnp.sum(gn * -jnp.log(1.0 - rn))
    l1 = -float(np.log(np.float32(0.001)))
    l0 = -float(np.log(np.float32(1.0) - np.float32(0.001)))
    unk = l1 * jnp.sum(1.0 - gp) + l0 * jnp.sum(1.0 - gn)
    gu = -(jnp.sum(gp * jnp.log(gp) + (1.0 - gp) * jnp.log(1.0 - gp))
           + jnp.sum(gn * jnp.log(gn) + (1.0 - gn) * jnp.log(1.0 - gn)))

    @pl.when(i == 0)
    def _():
        acc[0] = mf
        acc[1] = unk
        acc[2] = gu

    @pl.when(i > 0)
    def _():
        acc[0] += mf
        acc[1] += unk
        acc[2] += gu

    @pl.when(i == 3)
    def _():
        rl1 = 0.5 * sq_ref[0, 0] / float(B)
        rl2 = 0.5 * s2_ref[0, 0] / float(NI)
        rl3 = 0.5 * s3_ref[0, 0] / float(NU)
        reg = WD * (rl1 + rl3) + 0.1 * rl2
        inv = 1.0 / float(2 * B)
        out_ref[...] = (acc[0] * inv + 0.1 * (acc[1] * inv - acc[2] * inv)
                        + reg).reshape(1, 1)


_tc3 = pl.pallas_call(
    _tc3_body,
    grid=(4,),
    in_specs=[pl.BlockSpec((4096, 1), lambda i: (i, 0)),
              pl.BlockSpec((4096, 1), lambda i: (i, 0)),
              pl.BlockSpec((4096, K), lambda i: (i, 0)),
              pl.BlockSpec((4096, K), lambda i: (i, 0)),
              pl.BlockSpec((4096, K), lambda i: (i, 0)),
              pl.BlockSpec((1, 1), lambda i: (0, 0)),
              pl.BlockSpec((1, 1), lambda i: (0, 0)),
              pl.BlockSpec((1, 1), lambda i: (0, 0))],
    out_specs=pl.BlockSpec((1, 1), lambda i: (0, 0)),
    out_shape=jax.ShapeDtypeStruct((1, 1), jnp.float32),
    scratch_shapes=[pltpu.SMEM((4,), jnp.float32)],
)


def kernel(users, positive_items, negative_items, edge_index, edge_values,
           user_embedding, item_embedding, theta_user, w1, w2):
    users = users.astype(jnp.int32)
    positive_items = positive_items.astype(jnp.int32)
    negative_items = negative_items.astype(jnp.int32)
    edge_index = edge_index.astype(jnp.int32)

    theta, s3, s2 = _tc1(theta_user, w1.reshape(8, NU // 8),
                         w2.reshape(8, NI // 8))

    pad = EP - E
    rows2d = jnp.concatenate(
        [edge_index[0], jnp.zeros((pad,), dtype=jnp.int32)]).reshape(CR, 128)
    cols2d = jnp.concatenate(
        [edge_index[1], jnp.zeros((pad,), dtype=jnp.int32)]).reshape(CR, 128)
    zeros_z = jnp.zeros((ZCH, K), jnp.float32)
    users2d = users.reshape(BCR, 128)
    pos2d = positive_items.reshape(BCR, 128)
    neg2d = negative_items.reshape(BCR, 128)

    z_parts, u, p, n, thu = _sc_a(rows2d, cols2d, theta, user_embedding,
                                  item_embedding, users2d, pos2d, neg2d,
                                  zeros_z)

    scale = edge_values[0].reshape(1, 1)
    z1 = _tc2a(z_parts[0, :NI], z_parts[1, :NI], w1, w2, scale)
    ps, ns, sq = _tc2b(u, p, n)
    z1p, z1n = _sc_b(z1, pos2d, neg2d)
    loss = _tc3(ps, ns, thu, z1p, z1n, s2, s3, sq)
    return loss.reshape(())


# speedup vs baseline: 8.1454x; 8.1320x over previous
import jax
import jax.numpy as jnp
import numpy as np
from jax import lax
from jax.experimental import pallas as pl
from jax.experimental.pallas import tpu as pltpu
from jax.experimental.pallas import tpu_sc as plsc

NU = 25000
NI = 25000
N = NU + NI
K = 32
EMB = 128
E = 1600000
B = 16384
WD = 1e-4

NC, NS = 2, 16
NW = NC * NS
TP = 25088
EP = 1638400
CR = EP // 128
ECH = 32
NCH = CR // ECH
BCR = B // 128

_MESH = plsc.VectorSubcoreMesh(core_axis_name="c", subcore_axis_name="s",
                               num_cores=NC, num_subcores=NS)
_SC_PARAMS = pltpu.CompilerParams(use_tc_tiling_on_sc=False,
                                  needs_layout_passes=False)


def _tc_pack_body(r_ref, c_ref, out_ref):
    r = r_ref[...]
    c = c_ref[...]
    keep = (r >= NU) & (c < NU)
    out_ref[...] = jnp.where(keep, (r - NU) * 65536 + c,
                             jnp.full_like(r, -1))


_tc_pack = pl.pallas_call(
    _tc_pack_body,
    grid=(8,),
    in_specs=[pl.BlockSpec((CR // 8, 128), lambda i: (i, 0))] * 2,
    out_specs=pl.BlockSpec((CR // 8, 128), lambda i: (i, 0)),
    out_shape=jax.ShapeDtypeStruct((CR, 128), jnp.int32),
)


def _tc_prep_body(ue_ref, ie_ref, tu_ref, w1_ref, w2_ref,
                  ueT_ref, ieT_ref, thT_ref, s3_ref, s2_ref):
    i = pl.program_id(0)
    ueT_ref[...] = ue_ref[...].T
    ieT_ref[...] = ie_ref[...].T
    x = tu_ref[...]
    m = jnp.max(x, axis=1, keepdims=True)
    e = jnp.exp(x - m)
    th = e / jnp.sum(e, axis=1, keepdims=True)
    thT_ref[...] = th.T
    part = jnp.sum(x * x)

    @pl.when(i == 0)
    def _():
        s3_ref[...] = part.reshape(1, 1)
        w1 = w1_ref[...]
        w2 = w2_ref[...]
        s2_ref[...] = (jnp.sum(w1 * w1) + jnp.sum(w2 * w2)).reshape(1, 1)

    @pl.when(i > 0)
    def _():
        s3_ref[...] += part.reshape(1, 1)


_BLK = TP // 7


_tc_prep = pl.pallas_call(
    _tc_prep_body,
    grid=(7,),
    in_specs=[pl.BlockSpec((_BLK, EMB), lambda i: (i, 0)),
              pl.BlockSpec((_BLK, EMB), lambda i: (i, 0)),
              pl.BlockSpec((_BLK, K), lambda i: (i, 0)),
              pl.BlockSpec((8, NU // 8), lambda i: (0, 0)),
              pl.BlockSpec((8, NI // 8), lambda i: (0, 0))],
    out_specs=(pl.BlockSpec((EMB, _BLK), lambda i: (0, i)),
               pl.BlockSpec((EMB, _BLK), lambda i: (0, i)),
               pl.BlockSpec((K, _BLK), lambda i: (0, i)),
               pl.BlockSpec((1, 1), lambda i: (0, 0)),
               pl.BlockSpec((1, 1), lambda i: (0, 0))),
    out_shape=(jax.ShapeDtypeStruct((EMB, TP), jnp.float32),
               jax.ShapeDtypeStruct((EMB, TP), jnp.float32),
               jax.ShapeDtypeStruct((K, TP), jnp.float32),
               jax.ShapeDtypeStruct((1, 1), jnp.float32),
               jax.ShapeDtypeStruct((1, 1), jnp.float32)),
)


def _sc_edge_body(packed, thT, users2d, zT_out, thuT_out,
                  thbuf, acc, pk0, pk1, ubuf, obuf, sem0, sem1, sem2):
    c = lax.axis_index("c")
    s = lax.axis_index("s")
    wid = c * NS + s

    pltpu.sync_copy(thT.at[wid], thbuf)

    def zero(i, carry):
        acc[pl.ds(i * 16, 16)] = jnp.zeros((16,), jnp.float32)
        return carry

    lax.fori_loop(0, TP // 16, zero, 0)

    pltpu.async_copy(packed.at[pl.ds(0, ECH)], pk0, sem0)
    pltpu.async_copy(packed.at[pl.ds(ECH, ECH)], pk1, sem1)

    def chunk(i, carry):
        for h, (pk, sem) in enumerate(((pk0, sem0), (pk1, sem1))):
            g = 2 * i + h
            pltpu.make_async_copy(packed.at[pl.ds(0, ECH)], pk, sem).wait()

            def row(r, carry2):
                for l in range(8):
                    pkt = pk[r, pl.ds(l * 16, 16)]
                    keep = pkt >= 0
                    rn = jnp.where(keep, jnp.right_shift(pkt, 16), 0)
                    cn = jnp.where(keep, pkt & 0xFFFF, 0)
                    v = plsc.load_gather(thbuf, [cn], mask=keep)
                    plsc.addupdate_scatter(acc, [rn], v, mask=keep)
                return carry2

            lax.fori_loop(0, ECH, row, 0)

            @pl.when(g + 2 < NCH)
            def _():
                pltpu.async_copy(packed.at[pl.ds((g + 2) * ECH, ECH)], pk,
                                 sem)
        return carry

    lax.fori_loop(0, NCH // 2, chunk, 0)
    pltpu.sync_copy(acc, zT_out.at[wid])

    def thu(ci, carry):
        pltpu.sync_copy(users2d.at[pl.ds(ci * 8, 8)], ubuf)
        for r8 in range(8):
            for l in range(8):
                idx = ubuf[r8, pl.ds(l * 16, 16)]
                obuf[pl.ds(r8 * 128 + l * 16, 16)] = plsc.load_gather(
                    thbuf, [idx])
        pltpu.sync_copy(obuf, thuT_out.at[wid, pl.ds(ci * 1024, 1024)])
        return carry

    lax.fori_loop(0, B // 1024, thu, 0)


_sc_edge = pl.kernel(
    _sc_edge_body,
    out_type=(jax.ShapeDtypeStruct((K, TP), jnp.float32),
              jax.ShapeDtypeStruct((K, B), jnp.float32)),
    mesh=_MESH,
    scratch_types=[
        pltpu.VMEM((TP,), jnp.float32),
        pltpu.VMEM((TP,), jnp.float32),
        pltpu.VMEM((ECH, 128), jnp.int32),
        pltpu.VMEM((ECH, 128), jnp.int32),
        pltpu.VMEM((8, 128), jnp.int32),
        pltpu.VMEM((1024,), jnp.float32),
        pltpu.SemaphoreType.DMA,
        pltpu.SemaphoreType.DMA,
        pltpu.SemaphoreType.DMA,
    ],
    compiler_params=_SC_PARAMS,
)


def _sc_emb_body(ueT3, ieT3, users2d, pos2d, neg2d, uT, pT, nT,
                 tb0, tb1, tb2, tb3, idxb, ob, sem0):
    c = lax.axis_index("c")
    s = lax.axis_index("s")
    wid = c * NS + s
    tbs = (tb0, tb1, tb2, tb3)

    for tbl3, jobs in ((ueT3, ((users2d, uT),)),
                       (ieT3, ((pos2d, pT), (neg2d, nT)))):
        for cc in range(4):
            pltpu.sync_copy(tbl3.at[wid, cc], tbs[cc])
        for idx2d, out in jobs:
            def emb(ci, carry):
                pltpu.sync_copy(idx2d.at[pl.ds(ci * 8, 8)], idxb)
                for r8 in range(8):
                    for l in range(8):
                        sl = pl.ds(r8 * 128 + l * 16, 16)
                        idx = idxb[r8, pl.ds(l * 16, 16)]
                        for cc in range(4):
                            ob[cc, sl] = plsc.load_gather(tbs[cc], [idx])
                for cc in range(4):
                    pltpu.sync_copy(
                        ob.at[cc],
                        out.at[4 * wid + cc, pl.ds(ci * 1024, 1024)])
                return carry

            lax.fori_loop(0, B // 1024, emb, 0)


_sc_emb = pl.kernel(
    _sc_emb_body,
    out_type=(jax.ShapeDtypeStruct((EMB, B), jnp.float32),
              jax.ShapeDtypeStruct((EMB, B), jnp.float32),
              jax.ShapeDtypeStruct((EMB, B), jnp.float32)),
    mesh=_MESH,
    scratch_types=[
        pltpu.VMEM((TP,), jnp.float32),
        pltpu.VMEM((TP,), jnp.float32),
        pltpu.VMEM((TP,), jnp.float32),
        pltpu.VMEM((TP,), jnp.float32),
        pltpu.VMEM((8, 128), jnp.int32),
        pltpu.VMEM((4, 1024), jnp.float32),
        pltpu.SemaphoreType.DMA,
    ],
    compiler_params=_SC_PARAMS,
)


def _sc_z1_body(z1T, pos2d, neg2d, z1pT, z1nT, zrow, idxb, ob, sem0):
    c = lax.axis_index("c")
    s = lax.axis_index("s")
    wid = c * NS + s
    pltpu.sync_copy(z1T.at[wid], zrow)
    for idx2d, out in ((pos2d, z1pT), (neg2d, z1nT)):
        def gth(ci, carry):
            pltpu.sync_copy(idx2d.at[pl.ds(ci * 8, 8)], idxb)
            for r8 in range(8):
                for l in range(8):
                    idx = idxb[r8, pl.ds(l * 16, 16)]
                    ob[pl.ds(r8 * 128 + l * 16, 16)] = plsc.load_gather(
                        zrow, [idx])
            pltpu.sync_copy(ob, out.at[wid, pl.ds(ci * 1024, 1024)])
            return carry

        lax.fori_loop(0, B // 1024, gth, 0)


_sc_z1 = pl.kernel(
    _sc_z1_body,
    out_type=(jax.ShapeDtypeStruct((K, B), jnp.float32),
              jax.ShapeDtypeStruct((K, B), jnp.float32)),
    mesh=_MESH,
    scratch_types=[
        pltpu.VMEM((TP,), jnp.float32),
        pltpu.VMEM((8, 128), jnp.int32),
        pltpu.VMEM((1024,), jnp.float32),
        pltpu.SemaphoreType.DMA,
    ],
    compiler_params=_SC_PARAMS,
)


def _tc2a_body(z_ref, w1_ref, w2_ref, sc_ref, out_ref):
    t = z_ref[...] * sc_ref[0, 0] * w1_ref[...] + w2_ref[...]
    out_ref[...] = 1.0 / (1.0 + jnp.exp(-t))


_tc2a = pl.pallas_call(
    _tc2a_body,
    out_shape=jax.ShapeDtypeStruct((K, TP), jnp.float32),
)


def _tc2b_body(u_ref, p_ref, n_ref, ps_ref, ns_ref, sq_ref):
    i = pl.program_id(0)
    u = u_ref[...]
    p = p_ref[...]
    n = n_ref[...]
    ps_ref[...] = jnp.sum(u * p, axis=0, keepdims=True)
    ns_ref[...] = jnp.sum(u * n, axis=0, keepdims=True)
    acc = jnp.sum(u * u) + jnp.sum(p * p) + jnp.sum(n * n)

    @pl.when(i == 0)
    def _():
        sq_ref[...] = acc.reshape(1, 1)

    @pl.when(i > 0)
    def _():
        sq_ref[...] += acc.reshape(1, 1)


_tc2b = pl.pallas_call(
    _tc2b_body,
    grid=(8,),
    in_specs=[pl.BlockSpec((EMB, B // 8), lambda i: (0, i))] * 3,
    out_specs=(pl.BlockSpec((1, B // 8), lambda i: (0, i)),
               pl.BlockSpec((1, B // 8), lambda i: (0, i)),
               pl.BlockSpec((1, 1), lambda i: (0, 0))),
    out_shape=(jax.ShapeDtypeStruct((1, B), jnp.float32),
               jax.ShapeDtypeStruct((1, B), jnp.float32),
               jax.ShapeDtypeStruct((1, 1), jnp.float32)),
)


def _tc3_body(ps_ref, ns_ref, thu_ref, z1p_ref, z1n_ref, s2_ref, s3_ref,
              sq_ref, out_ref, acc):
    i = pl.program_id(0)
    thu = thu_ref[...]
    gp = jnp.sum(thu * z1p_ref[...], axis=0, keepdims=True)
    gn = jnp.sum(thu * z1n_ref[...], axis=0, keepdims=True)
    rp = 1.0 / (1.0 + jnp.exp(-ps_ref[...]))
    rn = 1.0 / (1.0 + jnp.exp(-ns_ref[...]))
    mf = jnp.sum(gp * -jnp.log(rp)) + jnp.sum(gn * -jnp.log(1.0 - rn))
    l1 = -float(np.log(np.float32(0.001)))
    l0 = -float(np.log(np.float32(1.0) - np.float32(0.001)))
    unk = l1 * jnp.sum(1.0 - gp) + l0 * jnp.sum(1.0 - gn)
    gu = -(jnp.sum(gp * jnp.log(gp) + (1.0 - gp) * jnp.log(1.0 - gp))
           + jnp.sum(gn * jnp.log(gn) + (1.0 - gn) * jnp.log(1.0 - gn)))

    @pl.when(i == 0)
    def _():
        acc[0] = mf
        acc[1] = unk
        acc[2] = gu

    @pl.when(i > 0)
    def _():
        acc[0] += mf
        acc[1] += unk
        acc[2] += gu

    @pl.when(i == 7)
    def _():
        rl1 = 0.5 * sq_ref[0, 0] / float(B)
        rl2 = 0.5 * s2_ref[0, 0] / float(NI)
        rl3 = 0.5 * s3_ref[0, 0] / float(NU)
        reg = WD * (rl1 + rl3) + 0.1 * rl2
        inv = 1.0 / float(2 * B)
        out_ref[...] = (acc[0] * inv + 0.1 * (acc[1] * inv - acc[2] * inv)
                        + reg).reshape(1, 1)


_tc3 = pl.pallas_call(
    _tc3_body,
    grid=(8,),
    in_specs=[pl.BlockSpec((1, B // 8), lambda i: (0, i)),
              pl.BlockSpec((1, B // 8), lambda i: (0, i)),
              pl.BlockSpec((K, B // 8), lambda i: (0, i)),
              pl.BlockSpec((K, B // 8), lambda i: (0, i)),
              pl.BlockSpec((K, B // 8), lambda i: (0, i)),
              pl.BlockSpec((1, 1), lambda i: (0, 0)),
              pl.BlockSpec((1, 1), lambda i: (0, 0)),
              pl.BlockSpec((1, 1), lambda i: (0, 0))],
    out_specs=pl.BlockSpec((1, 1), lambda i: (0, 0)),
    out_shape=jax.ShapeDtypeStruct((1, 1), jnp.float32),
    scratch_shapes=[pltpu.SMEM((4,), jnp.float32)],
)


def kernel(users, positive_items, negative_items, edge_index, edge_values,
           user_embedding, item_embedding, theta_user, w1, w2):
    users = users.astype(jnp.int32)
    positive_items = positive_items.astype(jnp.int32)
    negative_items = negative_items.astype(jnp.int32)
    edge_index = edge_index.astype(jnp.int32)

    pad = EP - E
    rows2d = jnp.concatenate(
        [edge_index[0], jnp.zeros((pad,), dtype=jnp.int32)]).reshape(CR, 128)
    cols2d = jnp.concatenate(
        [edge_index[1], jnp.zeros((pad,), dtype=jnp.int32)]).reshape(CR, 128)
    packed = _tc_pack(rows2d, cols2d)

    zpad = jnp.zeros((TP - NU, EMB), jnp.float32)
    uep = jnp.concatenate([user_embedding, zpad])
    iep = jnp.concatenate([item_embedding, zpad])
    tup = jnp.concatenate([theta_user, jnp.zeros((TP - NU, K), jnp.float32)])
    ueT, ieT, thT, s3, s2 = _tc_prep(uep, iep, tup,
                                     w1.reshape(8, NU // 8),
                                     w2.reshape(8, NI // 8))

    users2d = users.reshape(BCR, 128)
    pos2d = positive_items.reshape(BCR, 128)
    neg2d = negative_items.reshape(BCR, 128)

    zT, thuT = _sc_edge(packed, thT, users2d)
    uT, pT, nT = _sc_emb(ueT.reshape(NW, 4, TP), ieT.reshape(NW, 4, TP),
                         users2d, pos2d, neg2d)

    scale = edge_values[0].reshape(1, 1)
    w1p = jnp.pad(w1.reshape(1, NU), ((0, 0), (0, TP - NU)))
    w2p = jnp.pad(w2.reshape(1, NI), ((0, 0), (0, TP - NU)))
    z1T = _tc2a(zT, w1p, w2p, scale)
    z1pT, z1nT = _sc_z1(z1T, pos2d, neg2d)

    ps, ns, sq = _tc2b(uT, pT, nT)
    loss = _tc3(ps, ns, thuT, z1pT, z1nT, s2, s3, sq)
    return loss.reshape(())


# speedup vs baseline: 8.3862x; 1.0296x over previous
import jax
import jax.numpy as jnp
import numpy as np
from jax import lax
from jax.experimental import pallas as pl
from jax.experimental.pallas import tpu as pltpu
from jax.experimental.pallas import tpu_sc as plsc

NU = 25000
NI = 25000
N = NU + NI
K = 32
EMB = 128
E = 1600000
B = 16384
WD = 1e-4

NC, NS = 2, 16
NW = NC * NS
TP = 25088
EP = 1638400
CR = EP // 128
ECH = 32
NCH = CR // ECH
BCR = B // 128

_MESH = plsc.VectorSubcoreMesh(core_axis_name="c", subcore_axis_name="s",
                               num_cores=NC, num_subcores=NS)
_SC_PARAMS = pltpu.CompilerParams(use_tc_tiling_on_sc=False,
                                  needs_layout_passes=False)


def _tc_pack_body(r_ref, c_ref, out_ref):
    r = r_ref[...]
    c = c_ref[...]
    keep = (r >= NU) & (c < NU)
    out_ref[...] = jnp.where(keep, (r - NU) * 65536 + c,
                             jnp.full_like(r, -1))


_tc_pack = pl.pallas_call(
    _tc_pack_body,
    grid=(8,),
    in_specs=[pl.BlockSpec((CR // 8, 128), lambda i: (i, 0))] * 2,
    out_specs=pl.BlockSpec((CR // 8, 128), lambda i: (i, 0)),
    out_shape=jax.ShapeDtypeStruct((CR, 128), jnp.int32),
)


def _tc_prep_body(ue_ref, ie_ref, tu_ref, w1_ref, w2_ref,
                  ueT_ref, ieT_ref, thT_ref, s3_ref, s2_ref):
    i = pl.program_id(0)
    ueT_ref[...] = ue_ref[...].T
    ieT_ref[...] = ie_ref[...].T
    x = tu_ref[...]
    m = jnp.max(x, axis=1, keepdims=True)
    e = jnp.exp(x - m)
    th = e / jnp.sum(e, axis=1, keepdims=True)
    thT_ref[...] = th.T
    part = jnp.sum(x * x)

    @pl.when(i == 0)
    def _():
        s3_ref[...] = part.reshape(1, 1)
        w1 = w1_ref[...]
        w2 = w2_ref[...]
        s2_ref[...] = (jnp.sum(w1 * w1) + jnp.sum(w2 * w2)).reshape(1, 1)

    @pl.when(i > 0)
    def _():
        s3_ref[...] += part.reshape(1, 1)


_BLK = TP // 7


_tc_prep = pl.pallas_call(
    _tc_prep_body,
    grid=(7,),
    in_specs=[pl.BlockSpec((_BLK, EMB), lambda i: (i, 0)),
              pl.BlockSpec((_BLK, EMB), lambda i: (i, 0)),
              pl.BlockSpec((_BLK, K), lambda i: (i, 0)),
              pl.BlockSpec((8, NU // 8), lambda i: (0, 0)),
              pl.BlockSpec((8, NI // 8), lambda i: (0, 0))],
    out_specs=(pl.BlockSpec((EMB, _BLK), lambda i: (0, i)),
               pl.BlockSpec((EMB, _BLK), lambda i: (0, i)),
               pl.BlockSpec((K, _BLK), lambda i: (0, i)),
               pl.BlockSpec((1, 1), lambda i: (0, 0)),
               pl.BlockSpec((1, 1), lambda i: (0, 0))),
    out_shape=(jax.ShapeDtypeStruct((EMB, TP), jnp.float32),
               jax.ShapeDtypeStruct((EMB, TP), jnp.float32),
               jax.ShapeDtypeStruct((K, TP), jnp.float32),
               jax.ShapeDtypeStruct((1, 1), jnp.float32),
               jax.ShapeDtypeStruct((1, 1), jnp.float32)),
)


def _sc_edge_body(packed, thT, users2d, zT_out, thuT_out,
                  thbuf, acc, pk0, pk1, ubuf, obuf, sem0, sem1, sem2):
    c = lax.axis_index("c")
    s = lax.axis_index("s")
    wid = c * NS + s

    pltpu.sync_copy(thT.at[wid], thbuf)

    def zero(i, carry):
        acc[pl.ds(i * 16, 16)] = jnp.zeros((16,), jnp.float32)
        return carry

    lax.fori_loop(0, TP // 16, zero, 0)

    pltpu.async_copy(packed.at[pl.ds(0, ECH)], pk0, sem0)
    pltpu.async_copy(packed.at[pl.ds(ECH, ECH)], pk1, sem1)

    def chunk(i, carry):
        for h, (pk, sem) in enumerate(((pk0, sem0), (pk1, sem1))):
            g = 2 * i + h
            pltpu.make_async_copy(packed.at[pl.ds(0, ECH)], pk, sem).wait()

            def row(r, carry2):
                for l in range(8):
                    pkt = pk[r, pl.ds(l * 16, 16)]
                    keep = pkt >= 0
                    rn = jnp.right_shift(pkt, 16)
                    cn = pkt & 0x7FFF
                    v = plsc.load_gather(thbuf, [cn], mask=keep)
                    plsc.addupdate_scatter(acc, [rn], v, mask=keep)
                return carry2

            lax.fori_loop(0, ECH, row, 0)

            @pl.when(g + 2 < NCH)
            def _():
                pltpu.async_copy(packed.at[pl.ds((g + 2) * ECH, ECH)], pk,
                                 sem)
        return carry

    lax.fori_loop(0, NCH // 2, chunk, 0)
    pltpu.sync_copy(acc, zT_out.at[wid])

    def thu(ci, carry):
        pltpu.sync_copy(users2d.at[pl.ds(ci * 8, 8)], ubuf)
        for r8 in range(8):
            for l in range(8):
                idx = ubuf[r8, pl.ds(l * 16, 16)]
                obuf[pl.ds(r8 * 128 + l * 16, 16)] = plsc.load_gather(
                    thbuf, [idx])
        pltpu.sync_copy(obuf, thuT_out.at[wid, pl.ds(ci * 1024, 1024)])
        return carry

    lax.fori_loop(0, B // 1024, thu, 0)


_sc_edge = pl.kernel(
    _sc_edge_body,
    out_type=(jax.ShapeDtypeStruct((K, TP), jnp.float32),
              jax.ShapeDtypeStruct((K, B), jnp.float32)),
    mesh=_MESH,
    scratch_types=[
        pltpu.VMEM((TP,), jnp.float32),
        pltpu.VMEM((TP,), jnp.float32),
        pltpu.VMEM((ECH, 128), jnp.int32),
        pltpu.VMEM((ECH, 128), jnp.int32),
        pltpu.VMEM((8, 128), jnp.int32),
        pltpu.VMEM((1024,), jnp.float32),
        pltpu.SemaphoreType.DMA,
        pltpu.SemaphoreType.DMA,
        pltpu.SemaphoreType.DMA,
    ],
    compiler_params=_SC_PARAMS,
)


def _sc_emb_body(ueT3, ieT3, users2d, pos2d, neg2d, uT, pT, nT,
                 tb0, tb1, tb2, tb3, idxb, ob, sem0):
    c = lax.axis_index("c")
    s = lax.axis_index("s")
    wid = c * NS + s
    tbs = (tb0, tb1, tb2, tb3)

    for tbl3, jobs in ((ueT3, ((users2d, uT),)),
                       (ieT3, ((pos2d, pT), (neg2d, nT)))):
        for cc in range(4):
            pltpu.sync_copy(tbl3.at[wid, cc], tbs[cc])
        for idx2d, out in jobs:
            def emb(ci, carry):
                pltpu.sync_copy(idx2d.at[pl.ds(ci * 8, 8)], idxb)
                for r8 in range(8):
                    for l in range(8):
                        sl = pl.ds(r8 * 128 + l * 16, 16)
                        idx = idxb[r8, pl.ds(l * 16, 16)]
                        for cc in range(4):
                            ob[cc, sl] = plsc.load_gather(tbs[cc], [idx])
                for cc in range(4):
                    pltpu.sync_copy(
                        ob.at[cc],
                        out.at[4 * wid + cc, pl.ds(ci * 1024, 1024)])
                return carry

            lax.fori_loop(0, B // 1024, emb, 0)


_sc_emb = pl.kernel(
    _sc_emb_body,
    out_type=(jax.ShapeDtypeStruct((EMB, B), jnp.float32),
              jax.ShapeDtypeStruct((EMB, B), jnp.float32),
              jax.ShapeDtypeStruct((EMB, B), jnp.float32)),
    mesh=_MESH,
    scratch_types=[
        pltpu.VMEM((TP,), jnp.float32),
        pltpu.VMEM((TP,), jnp.float32),
        pltpu.VMEM((TP,), jnp.float32),
        pltpu.VMEM((TP,), jnp.float32),
        pltpu.VMEM((8, 128), jnp.int32),
        pltpu.VMEM((4, 1024), jnp.float32),
        pltpu.SemaphoreType.DMA,
    ],
    compiler_params=_SC_PARAMS,
)


def _sc_z1_body(z1T, pos2d, neg2d, z1pT, z1nT, zrow, idxb, ob, sem0):
    c = lax.axis_index("c")
    s = lax.axis_index("s")
    wid = c * NS + s
    pltpu.sync_copy(z1T.at[wid], zrow)
    for idx2d, out in ((pos2d, z1pT), (neg2d, z1nT)):
        def gth(ci, carry):
            pltpu.sync_copy(idx2d.at[pl.ds(ci * 8, 8)], idxb)
            for r8 in range(8):
                for l in range(8):
                    idx = idxb[r8, pl.ds(l * 16, 16)]
                    ob[pl.ds(r8 * 128 + l * 16, 16)] = plsc.load_gather(
                        zrow, [idx])
            pltpu.sync_copy(ob, out.at[wid, pl.ds(ci * 1024, 1024)])
            return carry

        lax.fori_loop(0, B // 1024, gth, 0)


_sc_z1 = pl.kernel(
    _sc_z1_body,
    out_type=(jax.ShapeDtypeStruct((K, B), jnp.float32),
              jax.ShapeDtypeStruct((K, B), jnp.float32)),
    mesh=_MESH,
    scratch_types=[
        pltpu.VMEM((TP,), jnp.float32),
        pltpu.VMEM((8, 128), jnp.int32),
        pltpu.VMEM((1024,), jnp.float32),
        pltpu.SemaphoreType.DMA,
    ],
    compiler_params=_SC_PARAMS,
)


def _tc2a_body(z_ref, w1_ref, w2_ref, sc_ref, out_ref):
    t = z_ref[...] * sc_ref[0, 0] * w1_ref[...] + w2_ref[...]
    out_ref[...] = 1.0 / (1.0 + jnp.exp(-t))


_tc2a = pl.pallas_call(
    _tc2a_body,
    out_shape=jax.ShapeDtypeStruct((K, TP), jnp.float32),
)


def _tc2b_body(u_ref, p_ref, n_ref, ps_ref, ns_ref, sq_ref):
    i = pl.program_id(0)
    u = u_ref[...]
    p = p_ref[...]
    n = n_ref[...]
    ps_ref[...] = jnp.sum(u * p, axis=0, keepdims=True)
    ns_ref[...] = jnp.sum(u * n, axis=0, keepdims=True)
    acc = jnp.sum(u * u) + jnp.sum(p * p) + jnp.sum(n * n)

    @pl.when(i == 0)
    def _():
        sq_ref[...] = acc.reshape(1, 1)

    @pl.when(i > 0)
    def _():
        sq_ref[...] += acc.reshape(1, 1)


_tc2b = pl.pallas_call(
    _tc2b_body,
    grid=(8,),
    in_specs=[pl.BlockSpec((EMB, B // 8), lambda i: (0, i))] * 3,
    out_specs=(pl.BlockSpec((1, B // 8), lambda i: (0, i)),
               pl.BlockSpec((1, B // 8), lambda i: (0, i)),
               pl.BlockSpec((1, 1), lambda i: (0, 0))),
    out_shape=(jax.ShapeDtypeStruct((1, B), jnp.float32),
               jax.ShapeDtypeStruct((1, B), jnp.float32),
               jax.ShapeDtypeStruct((1, 1), jnp.float32)),
)


def _tc3_body(ps_ref, ns_ref, thu_ref, z1p_ref, z1n_ref, s2_ref, s3_ref,
              sq_ref, out_ref, acc):
    i = pl.program_id(0)
    thu = thu_ref[...]
    gp = jnp.sum(thu * z1p_ref[...], axis=0, keepdims=True)
    gn = jnp.sum(thu * z1n_ref[...], axis=0, keepdims=True)
    rp = 1.0 / (1.0 + jnp.exp(-ps_ref[...]))
    rn = 1.0 / (1.0 + jnp.exp(-ns_ref[...]))
    mf = jnp.sum(gp * -jnp.log(rp)) + jnp.sum(gn * -jnp.log(1.0 - rn))
    l1 = -float(np.log(np.float32(0.001)))
    l0 = -float(np.log(np.float32(1.0) - np.float32(0.001)))
    unk = l1 * jnp.sum(1.0 - gp) + l0 * jnp.sum(1.0 - gn)
    gu = -(jnp.sum(gp * jnp.log(gp) + (1.0 - gp) * jnp.log(1.0 - gp))
           + jnp.sum(gn * jnp.log(gn) + (1.0 - gn) * jnp.log(1.0 - gn)))

    @pl.when(i == 0)
    def _():
        acc[0] = mf
        acc[1] = unk
        acc[2] = gu

    @pl.when(i > 0)
    def _():
        acc[0] += mf
        acc[1] += unk
        acc[2] += gu

    @pl.when(i == 7)
    def _():
        rl1 = 0.5 * sq_ref[0, 0] / float(B)
        rl2 = 0.5 * s2_ref[0, 0] / float(NI)
        rl3 = 0.5 * s3_ref[0, 0] / float(NU)
        reg = WD * (rl1 + rl3) + 0.1 * rl2
        inv = 1.0 / float(2 * B)
        out_ref[...] = (acc[0] * inv + 0.1 * (acc[1] * inv - acc[2] * inv)
                        + reg).reshape(1, 1)


_tc3 = pl.pallas_call(
    _tc3_body,
    grid=(8,),
    in_specs=[pl.BlockSpec((1, B // 8), lambda i: (0, i)),
              pl.BlockSpec((1, B // 8), lambda i: (0, i)),
              pl.BlockSpec((K, B // 8), lambda i: (0, i)),
              pl.BlockSpec((K, B // 8), lambda i: (0, i)),
              pl.BlockSpec((K, B // 8), lambda i: (0, i)),
              pl.BlockSpec((1, 1), lambda i: (0, 0)),
              pl.BlockSpec((1, 1), lambda i: (0, 0)),
              pl.BlockSpec((1, 1), lambda i: (0, 0))],
    out_specs=pl.BlockSpec((1, 1), lambda i: (0, 0)),
    out_shape=jax.ShapeDtypeStruct((1, 1), jnp.float32),
    scratch_shapes=[pltpu.SMEM((4,), jnp.float32)],
)


def kernel(users, positive_items, negative_items, edge_index, edge_values,
           user_embedding, item_embedding, theta_user, w1, w2):
    users = users.astype(jnp.int32)
    positive_items = positive_items.astype(jnp.int32)
    negative_items = negative_items.astype(jnp.int32)
    edge_index = edge_index.astype(jnp.int32)

    pad = EP - E
    rows2d = jnp.concatenate(
        [edge_index[0], jnp.zeros((pad,), dtype=jnp.int32)]).reshape(CR, 128)
    cols2d = jnp.concatenate(
        [edge_index[1], jnp.zeros((pad,), dtype=jnp.int32)]).reshape(CR, 128)
    packed = _tc_pack(rows2d, cols2d)

    zpad = jnp.zeros((TP - NU, EMB), jnp.float32)
    uep = jnp.concatenate([user_embedding, zpad])
    iep = jnp.concatenate([item_embedding, zpad])
    tup = jnp.concatenate([theta_user, jnp.zeros((TP - NU, K), jnp.float32)])
    ueT, ieT, thT, s3, s2 = _tc_prep(uep, iep, tup,
                                     w1.reshape(8, NU // 8),
                                     w2.reshape(8, NI // 8))

    users2d = users.reshape(BCR, 128)
    pos2d = positive_items.reshape(BCR, 128)
    neg2d = negative_items.reshape(BCR, 128)

    zT, thuT = _sc_edge(packed, thT, users2d)
    uT, pT, nT = _sc_emb(ueT.reshape(NW, 4, TP), ieT.reshape(NW, 4, TP),
                         users2d, pos2d, neg2d)

    scale = edge_values[0].reshape(1, 1)
    w1p = jnp.pad(w1.reshape(1, NU), ((0, 0), (0, TP - NU)))
    w2p = jnp.pad(w2.reshape(1, NI), ((0, 0), (0, TP - NU)))
    z1T = _tc2a(zT, w1p, w2p, scale)
    z1pT, z1nT = _sc_z1(z1T, pos2d, neg2d)

    ps, ns, sq = _tc2b(uT, pT, nT)
    loss = _tc3(ps, ns, thuT, z1pT, z1nT, s2, s3, sq)
    return loss.reshape(())


# speedup vs baseline: 16.4156x; 1.9575x over previous
import jax
import jax.numpy as jnp
import numpy as np
from jax import lax
from jax.experimental import pallas as pl
from jax.experimental.pallas import tpu as pltpu
from jax.experimental.pallas import tpu_sc as plsc

NU = 25000
NI = 25000
N = NU + NI
K = 32
EMB = 128
E = 1600000
B = 16384
WD = 1e-4

NC, NS = 2, 16
NW = NC * NS
TP = 25088
EP = 1638400
CR = EP // 128
ECH = 32
NCH = CR // ECH
BCR = B // 128

_MESH = plsc.VectorSubcoreMesh(core_axis_name="c", subcore_axis_name="s",
                               num_cores=NC, num_subcores=NS)
_SC_PARAMS = pltpu.CompilerParams(use_tc_tiling_on_sc=False,
                                  needs_layout_passes=False)


def _tc_pack_body(r_ref, c_ref, out_ref):
    r = r_ref[...]
    c = c_ref[...]
    keep = (r >= NU) & (c < NU)
    out_ref[...] = jnp.where(keep, (r - NU) * 65536 + c,
                             jnp.full_like(r, -1))


_tc_pack = pl.pallas_call(
    _tc_pack_body,
    grid=(8,),
    in_specs=[pl.BlockSpec((CR // 8, 128), lambda i: (i, 0))] * 2,
    out_specs=pl.BlockSpec((CR // 8, 128), lambda i: (i, 0)),
    out_shape=jax.ShapeDtypeStruct((CR, 128), jnp.int32),
)


def _tc_prep_body(ue_ref, ie_ref, tu_ref, w1_ref, w2_ref,
                  ueT_ref, ieT_ref, thT_ref, s3_ref, s2_ref):
    i = pl.program_id(0)
    ueT_ref[...] = ue_ref[...].T
    ieT_ref[...] = ie_ref[...].T
    x = tu_ref[...]
    m = jnp.max(x, axis=1, keepdims=True)
    e = jnp.exp(x - m)
    th = e / jnp.sum(e, axis=1, keepdims=True)
    thT_ref[...] = th.T
    part = jnp.sum(x * x)

    @pl.when(i == 0)
    def _():
        s3_ref[...] = part.reshape(1, 1)
        w1 = w1_ref[...]
        w2 = w2_ref[...]
        s2_ref[...] = (jnp.sum(w1 * w1) + jnp.sum(w2 * w2)).reshape(1, 1)

    @pl.when(i > 0)
    def _():
        s3_ref[...] += part.reshape(1, 1)


_BLK = TP // 7


_tc_prep = pl.pallas_call(
    _tc_prep_body,
    grid=(7,),
    in_specs=[pl.BlockSpec((_BLK, EMB), lambda i: (i, 0)),
              pl.BlockSpec((_BLK, EMB), lambda i: (i, 0)),
              pl.BlockSpec((_BLK, K), lambda i: (i, 0)),
              pl.BlockSpec((8, NU // 8), lambda i: (0, 0)),
              pl.BlockSpec((8, NI // 8), lambda i: (0, 0))],
    out_specs=(pl.BlockSpec((EMB, _BLK), lambda i: (0, i)),
               pl.BlockSpec((EMB, _BLK), lambda i: (0, i)),
               pl.BlockSpec((K, _BLK), lambda i: (0, i)),
               pl.BlockSpec((1, 1), lambda i: (0, 0)),
               pl.BlockSpec((1, 1), lambda i: (0, 0))),
    out_shape=(jax.ShapeDtypeStruct((EMB, TP), jnp.float32),
               jax.ShapeDtypeStruct((EMB, TP), jnp.float32),
               jax.ShapeDtypeStruct((K, TP), jnp.float32),
               jax.ShapeDtypeStruct((1, 1), jnp.float32),
               jax.ShapeDtypeStruct((1, 1), jnp.float32)),
)


def _sc_comp_body(packed, cpk_out, ccnt_out,
                  pend, pk0, pk1, cbuf, sem0, sem1):
    c = lax.axis_index("c")
    s = lax.axis_index("s")
    wid = c * NS + s
    base = wid * (CR // NW)

    pltpu.async_copy(packed.at[pl.ds(base, 16)], pk0, sem0)
    pltpu.async_copy(packed.at[pl.ds(base + 16, 16)], pk1, sem1)
    off = jnp.zeros((16,), jnp.int32)
    for g in range(25):
        pk, sem = (pk0, sem0) if g % 2 == 0 else (pk1, sem1)
        pltpu.make_async_copy(packed.at[pl.ds(base, 16)], pk, sem).wait()

        def row(r, ofv):
            for l in range(8):
                pkt = pk[r, pl.ds(l * 16, 16)]
                keep = pkt >= 0
                pos = ofv + plsc.cumsum(keep.astype(jnp.int32)) - 1
                plsc.store_scatter(pend, [pos], pkt, mask=keep)
                ofv = ofv + plsc.all_reduce_population_count(keep)
            return ofv

        off = lax.fori_loop(0, 16, row, off)
        if g + 2 < 25:
            pltpu.async_copy(packed.at[pl.ds(base + (g + 2) * 16, 16)], pk,
                             sem)
    pend_end = (off + 1023) & ~1023
    lanes = jnp.arange(16, dtype=jnp.int32)
    neg1 = jnp.full((16,), -1, jnp.int32)
    for j in range(64):
        pos = off + lanes + j * 16
        plsc.store_scatter(pend, [pos], neg1, mask=pos < pend_end)
    nch = jnp.right_shift(pend_end, 10)
    for j in range(8):
        cbuf[pl.ds(j * 16, 16)] = nch
    pltpu.sync_copy(cbuf, ccnt_out.at[wid])
    pltpu.sync_copy(pend, cpk_out.at[wid])


_sc_comp = pl.kernel(
    _sc_comp_body,
    out_type=(jax.ShapeDtypeStruct((NW, 51200), jnp.int32),
              jax.ShapeDtypeStruct((NW, 128), jnp.int32)),
    mesh=_MESH,
    scratch_types=[
        pltpu.VMEM((51200,), jnp.int32),
        pltpu.VMEM((16, 128), jnp.int32),
        pltpu.VMEM((16, 128), jnp.int32),
        pltpu.VMEM((128,), jnp.int32),
        pltpu.SemaphoreType.DMA,
        pltpu.SemaphoreType.DMA,
    ],
    compiler_params=_SC_PARAMS,
)


def _sc_edge_body(cpk, ccnt, thT, users2d, zT_out, thuT_out,
                  thbuf, acc, pk0, pk1, ubuf, obuf, cntb, sem0, sem1, sem2):
    c = lax.axis_index("c")
    s = lax.axis_index("s")
    wid = c * NS + s

    pltpu.sync_copy(thT.at[wid], thbuf)
    pltpu.sync_copy(ccnt, cntb)

    def zero(i, carry):
        acc[pl.ds(i * 16, 16)] = jnp.zeros((16,), jnp.float32)
        return carry

    lax.fori_loop(0, TP // 16, zero, 0)

    for w in range(NW):
        trip = cntb[w, pl.ds(0, 16)][0]

        @pl.when(trip > 0)
        def _():
            pltpu.async_copy(cpk.at[w, pl.ds(0, 1024)], pk0, sem0)

        @pl.when(trip > 1)
        def _():
            pltpu.async_copy(cpk.at[w, pl.ds(1024, 1024)], pk1, sem1)

        def duo(i, carry):
            for h, (pk, sem) in enumerate(((pk0, sem0), (pk1, sem1))):
                g = 2 * i + h

                @pl.when(g < trip)
                def _():
                    pltpu.make_async_copy(cpk.at[w, pl.ds(0, 1024)], pk,
                                          sem).wait()

                    def row(v, c2):
                        for l in range(4):
                            pkt = pk[pl.ds(v * 64 + l * 16, 16)]
                            keep = pkt >= 0
                            rn = jnp.right_shift(pkt, 16)
                            cn = pkt & 0x7FFF
                            vv = plsc.load_gather(thbuf, [cn], mask=keep)
                            plsc.addupdate_scatter(acc, [rn], vv, mask=keep)
                        return c2

                    lax.fori_loop(0, 16, row, 0)

                    @pl.when(g + 2 < trip)
                    def _():
                        pltpu.async_copy(
                            cpk.at[w, pl.ds((g + 2) * 1024, 1024)], pk, sem)
            return carry

        lax.fori_loop(0, (trip + 1) // 2, duo, 0)
    pltpu.sync_copy(acc, zT_out.at[wid])

    def thu(ci, carry):
        pltpu.sync_copy(users2d.at[pl.ds(ci * 8, 8)], ubuf)
        for r8 in range(8):
            for l in range(8):
                idx = ubuf[r8, pl.ds(l * 16, 16)]
                obuf[pl.ds(r8 * 128 + l * 16, 16)] = plsc.load_gather(
                    thbuf, [idx])
        pltpu.sync_copy(obuf, thuT_out.at[wid, pl.ds(ci * 1024, 1024)])
        return carry

    lax.fori_loop(0, B // 1024, thu, 0)


_sc_edge = pl.kernel(
    _sc_edge_body,
    out_type=(jax.ShapeDtypeStruct((K, TP), jnp.float32),
              jax.ShapeDtypeStruct((K, B), jnp.float32)),
    mesh=_MESH,
    scratch_types=[
        pltpu.VMEM((TP,), jnp.float32),
        pltpu.VMEM((TP,), jnp.float32),
        pltpu.VMEM((1024,), jnp.int32),
        pltpu.VMEM((1024,), jnp.int32),
        pltpu.VMEM((8, 128), jnp.int32),
        pltpu.VMEM((1024,), jnp.float32),
        pltpu.VMEM((NW, 128), jnp.int32),
        pltpu.SemaphoreType.DMA,
        pltpu.SemaphoreType.DMA,
        pltpu.SemaphoreType.DMA,
    ],
    compiler_params=_SC_PARAMS,
)


def _sc_emb_body(ueT3, ieT3, users2d, pos2d, neg2d, uT, pT, nT,
                 tb0, tb1, tb2, tb3, idxb, ob, sem0):
    c = lax.axis_index("c")
    s = lax.axis_index("s")
    wid = c * NS + s
    tbs = (tb0, tb1, tb2, tb3)

    for tbl3, jobs in ((ueT3, ((users2d, uT),)),
                       (ieT3, ((pos2d, pT), (neg2d, nT)))):
        for cc in range(4):
            pltpu.sync_copy(tbl3.at[wid, cc], tbs[cc])
        for idx2d, out in jobs:
            def emb(ci, carry):
                pltpu.sync_copy(idx2d.at[pl.ds(ci * 8, 8)], idxb)
                for r8 in range(8):
                    for l in range(8):
                        sl = pl.ds(r8 * 128 + l * 16, 16)
                        idx = idxb[r8, pl.ds(l * 16, 16)]
                        for cc in range(4):
                            ob[cc, sl] = plsc.load_gather(tbs[cc], [idx])
                for cc in range(4):
                    pltpu.sync_copy(
                        ob.at[cc],
                        out.at[4 * wid + cc, pl.ds(ci * 1024, 1024)])
                return carry

            lax.fori_loop(0, B // 1024, emb, 0)


_sc_emb = pl.kernel(
    _sc_emb_body,
    out_type=(jax.ShapeDtypeStruct((EMB, B), jnp.float32),
              jax.ShapeDtypeStruct((EMB, B), jnp.float32),
              jax.ShapeDtypeStruct((EMB, B), jnp.float32)),
    mesh=_MESH,
    scratch_types=[
        pltpu.VMEM((TP,), jnp.float32),
        pltpu.VMEM((TP,), jnp.float32),
        pltpu.VMEM((TP,), jnp.float32),
        pltpu.VMEM((TP,), jnp.float32),
        pltpu.VMEM((8, 128), jnp.int32),
        pltpu.VMEM((4, 1024), jnp.float32),
        pltpu.SemaphoreType.DMA,
    ],
    compiler_params=_SC_PARAMS,
)


def _sc_z1_body(z1T, pos2d, neg2d, z1pT, z1nT, zrow, idxb, ob, sem0):
    c = lax.axis_index("c")
    s = lax.axis_index("s")
    wid = c * NS + s
    pltpu.sync_copy(z1T.at[wid], zrow)
    for idx2d, out in ((pos2d, z1pT), (neg2d, z1nT)):
        def gth(ci, carry):
            pltpu.sync_copy(idx2d.at[pl.ds(ci * 8, 8)], idxb)
            for r8 in range(8):
                for l in range(8):
                    idx = idxb[r8, pl.ds(l * 16, 16)]
                    ob[pl.ds(r8 * 128 + l * 16, 16)] = plsc.load_gather(
                        zrow, [idx])
            pltpu.sync_copy(ob, out.at[wid, pl.ds(ci * 1024, 1024)])
            return carry

        lax.fori_loop(0, B // 1024, gth, 0)


_sc_z1 = pl.kernel(
    _sc_z1_body,
    out_type=(jax.ShapeDtypeStruct((K, B), jnp.float32),
              jax.ShapeDtypeStruct((K, B), jnp.float32)),
    mesh=_MESH,
    scratch_types=[
        pltpu.VMEM((TP,), jnp.float32),
        pltpu.VMEM((8, 128), jnp.int32),
        pltpu.VMEM((1024,), jnp.float32),
        pltpu.SemaphoreType.DMA,
    ],
    compiler_params=_SC_PARAMS,
)


def _tc2a_body(z_ref, w1_ref, w2_ref, sc_ref, out_ref):
    t = z_ref[...] * sc_ref[0, 0] * w1_ref[...] + w2_ref[...]
    out_ref[...] = 1.0 / (1.0 + jnp.exp(-t))


_tc2a = pl.pallas_call(
    _tc2a_body,
    out_shape=jax.ShapeDtypeStruct((K, TP), jnp.float32),
)


def _tc2b_body(u_ref, p_ref, n_ref, ps_ref, ns_ref, sq_ref):
    i = pl.program_id(0)
    u = u_ref[...]
    p = p_ref[...]
    n = n_ref[...]
    ps_ref[...] = jnp.sum(u * p, axis=0, keepdims=True)
    ns_ref[...] = jnp.sum(u * n, axis=0, keepdims=True)
    acc = jnp.sum(u * u) + jnp.sum(p * p) + jnp.sum(n * n)

    @pl.when(i == 0)
    def _():
        sq_ref[...] = acc.reshape(1, 1)

    @pl.when(i > 0)
    def _():
        sq_ref[...] += acc.reshape(1, 1)


_tc2b = pl.pallas_call(
    _tc2b_body,
    grid=(8,),
    in_specs=[pl.BlockSpec((EMB, B // 8), lambda i: (0, i))] * 3,
    out_specs=(pl.BlockSpec((1, B // 8), lambda i: (0, i)),
               pl.BlockSpec((1, B // 8), lambda i: (0, i)),
               pl.BlockSpec((1, 1), lambda i: (0, 0))),
    out_shape=(jax.ShapeDtypeStruct((1, B), jnp.float32),
               jax.ShapeDtypeStruct((1, B), jnp.float32),
               jax.ShapeDtypeStruct((1, 1), jnp.float32)),
)


def _tc3_body(ps_ref, ns_ref, thu_ref, z1p_ref, z1n_ref, s2_ref, s3_ref,
              sq_ref, out_ref, acc):
    i = pl.program_id(0)
    thu = thu_ref[...]
    gp = jnp.sum(thu * z1p_ref[...], axis=0, keepdims=True)
    gn = jnp.sum(thu * z1n_ref[...], axis=0, keepdims=True)
    rp = 1.0 / (1.0 + jnp.exp(-ps_ref[...]))
    rn = 1.0 / (1.0 + jnp.exp(-ns_ref[...]))
    mf = jnp.sum(gp * -jnp.log(rp)) + jnp.sum(gn * -jnp.log(1.0 - rn))
    l1 = -float(np.log(np.float32(0.001)))
    l0 = -float(np.log(np.float32(1.0) - np.float32(0.001)))
    unk = l1 * jnp.sum(1.0 - gp) + l0 * jnp.sum(1.0 - gn)
    gu = -(jnp.sum(gp * jnp.log(gp) + (1.0 - gp) * jnp.log(1.0 - gp))
           + jnp.sum(gn * jnp.log(gn) + (1.0 - gn) * jnp.log(1.0 - gn)))

    @pl.when(i == 0)
    def _():
        acc[0] = mf
        acc[1] = unk
        acc[2] = gu

    @pl.when(i > 0)
    def _():
        acc[0] += mf
        acc[1] += unk
        acc[2] += gu

    @pl.when(i == 7)
    def _():
        rl1 = 0.5 * sq_ref[0, 0] / float(B)
        rl2 = 0.5 * s2_ref[0, 0] / float(NI)
        rl3 = 0.5 * s3_ref[0, 0] / float(NU)
        reg = WD * (rl1 + rl3) + 0.1 * rl2
        inv = 1.0 / float(2 * B)
        out_ref[...] = (acc[0] * inv + 0.1 * (acc[1] * inv - acc[2] * inv)
                        + reg).reshape(1, 1)


_tc3 = pl.pallas_call(
    _tc3_body,
    grid=(8,),
    in_specs=[pl.BlockSpec((1, B // 8), lambda i: (0, i)),
              pl.BlockSpec((1, B // 8), lambda i: (0, i)),
              pl.BlockSpec((K, B // 8), lambda i: (0, i)),
              pl.BlockSpec((K, B // 8), lambda i: (0, i)),
              pl.BlockSpec((K, B // 8), lambda i: (0, i)),
              pl.BlockSpec((1, 1), lambda i: (0, 0)),
              pl.BlockSpec((1, 1), lambda i: (0, 0)),
              pl.BlockSpec((1, 1), lambda i: (0, 0))],
    out_specs=pl.BlockSpec((1, 1), lambda i: (0, 0)),
    out_shape=jax.ShapeDtypeStruct((1, 1), jnp.float32),
    scratch_shapes=[pltpu.SMEM((4,), jnp.float32)],
)


def kernel(users, positive_items, negative_items, edge_index, edge_values,
           user_embedding, item_embedding, theta_user, w1, w2):
    users = users.astype(jnp.int32)
    positive_items = positive_items.astype(jnp.int32)
    negative_items = negative_items.astype(jnp.int32)
    edge_index = edge_index.astype(jnp.int32)

    pad = EP - E
    rows2d = jnp.concatenate(
        [edge_index[0], jnp.zeros((pad,), dtype=jnp.int32)]).reshape(CR, 128)
    cols2d = jnp.concatenate(
        [edge_index[1], jnp.zeros((pad,), dtype=jnp.int32)]).reshape(CR, 128)
    packed = _tc_pack(rows2d, cols2d)
    cpk, ccnt = _sc_comp(packed)

    zpad = jnp.zeros((TP - NU, EMB), jnp.float32)
    uep = jnp.concatenate([user_embedding, zpad])
    iep = jnp.concatenate([item_embedding, zpad])
    tup = jnp.concatenate([theta_user, jnp.zeros((TP - NU, K), jnp.float32)])
    ueT, ieT, thT, s3, s2 = _tc_prep(uep, iep, tup,
                                     w1.reshape(8, NU // 8),
                                     w2.reshape(8, NI // 8))

    users2d = users.reshape(BCR, 128)
    pos2d = positive_items.reshape(BCR, 128)
    neg2d = negative_items.reshape(BCR, 128)

    zT, thuT = _sc_edge(cpk, ccnt, thT, users2d)
    uT, pT, nT = _sc_emb(ueT.reshape(NW, 4, TP), ieT.reshape(NW, 4, TP),
                         users2d, pos2d, neg2d)

    scale = edge_values[0].reshape(1, 1)
    w1p = jnp.pad(w1.reshape(1, NU), ((0, 0), (0, TP - NU)))
    w2p = jnp.pad(w2.reshape(1, NI), ((0, 0), (0, TP - NU)))
    z1T = _tc2a(zT, w1p, w2p, scale)
    z1pT, z1nT = _sc_z1(z1T, pos2d, neg2d)

    ps, ns, sq = _tc2b(uT, pT, nT)
    loss = _tc3(ps, ns, thuT, z1pT, z1nT, s2, s3, sq)
    return loss.reshape(())


# speedup vs baseline: 16.8593x; 1.0270x over previous
import jax
import jax.numpy as jnp
import numpy as np
from jax import lax
from jax.experimental import pallas as pl
from jax.experimental.pallas import tpu as pltpu
from jax.experimental.pallas import tpu_sc as plsc

NU = 25000
NI = 25000
N = NU + NI
K = 32
EMB = 128
E = 1600000
B = 16384
WD = 1e-4

NC, NS = 2, 16
NW = NC * NS
TP = 25088
EP = 1638400
CR = EP // 128
ECH = 32
NCH = CR // ECH
BCR = B // 128

_MESH = plsc.VectorSubcoreMesh(core_axis_name="c", subcore_axis_name="s",
                               num_cores=NC, num_subcores=NS)
_SC_PARAMS = pltpu.CompilerParams(use_tc_tiling_on_sc=False,
                                  needs_layout_passes=False)


def _tc_pack_body(r_ref, c_ref, out_ref):
    r = r_ref[...]
    c = c_ref[...]
    keep = (r >= NU) & (c < NU)
    out_ref[...] = jnp.where(keep, (r - NU) * 65536 + c,
                             jnp.full_like(r, -1))


_tc_pack = pl.pallas_call(
    _tc_pack_body,
    grid=(8,),
    in_specs=[pl.BlockSpec((CR // 8, 128), lambda i: (i, 0))] * 2,
    out_specs=pl.BlockSpec((CR // 8, 128), lambda i: (i, 0)),
    out_shape=jax.ShapeDtypeStruct((CR, 128), jnp.int32),
)


def _tc_prep_body(ue_ref, ie_ref, tu_ref, w1_ref, w2_ref,
                  ueT_ref, ieT_ref, thT_ref, s3_ref, s2_ref):
    i = pl.program_id(0)
    ueT_ref[...] = ue_ref[...].T
    ieT_ref[...] = ie_ref[...].T
    x = tu_ref[...]
    m = jnp.max(x, axis=1, keepdims=True)
    e = jnp.exp(x - m)
    th = e / jnp.sum(e, axis=1, keepdims=True)
    thT_ref[...] = th.T
    part = jnp.sum(x * x)

    @pl.when(i == 0)
    def _():
        s3_ref[...] = part.reshape(1, 1)
        w1 = w1_ref[...]
        w2 = w2_ref[...]
        s2_ref[...] = (jnp.sum(w1 * w1) + jnp.sum(w2 * w2)).reshape(1, 1)

    @pl.when(i > 0)
    def _():
        s3_ref[...] += part.reshape(1, 1)


_BLK = TP // 7


_tc_prep = pl.pallas_call(
    _tc_prep_body,
    grid=(7,),
    in_specs=[pl.BlockSpec((_BLK, EMB), lambda i: (i, 0)),
              pl.BlockSpec((_BLK, EMB), lambda i: (i, 0)),
              pl.BlockSpec((_BLK, K), lambda i: (i, 0)),
              pl.BlockSpec((8, NU // 8), lambda i: (0, 0)),
              pl.BlockSpec((8, NI // 8), lambda i: (0, 0))],
    out_specs=(pl.BlockSpec((EMB, _BLK), lambda i: (0, i)),
               pl.BlockSpec((EMB, _BLK), lambda i: (0, i)),
               pl.BlockSpec((K, _BLK), lambda i: (0, i)),
               pl.BlockSpec((1, 1), lambda i: (0, 0)),
               pl.BlockSpec((1, 1), lambda i: (0, 0))),
    out_shape=(jax.ShapeDtypeStruct((EMB, TP), jnp.float32),
               jax.ShapeDtypeStruct((EMB, TP), jnp.float32),
               jax.ShapeDtypeStruct((K, TP), jnp.float32),
               jax.ShapeDtypeStruct((1, 1), jnp.float32),
               jax.ShapeDtypeStruct((1, 1), jnp.float32)),
)


def _sc_comp_body(packed, cpk_out, ccnt_out,
                  pend, pk0, pk1, cbuf, sem0, sem1):
    c = lax.axis_index("c")
    s = lax.axis_index("s")
    wid = c * NS + s
    base = wid * (CR // NW)

    pltpu.async_copy(packed.at[pl.ds(base, 16)], pk0, sem0)
    pltpu.async_copy(packed.at[pl.ds(base + 16, 16)], pk1, sem1)
    off = jnp.zeros((16,), jnp.int32)
    for g in range(25):
        pk, sem = (pk0, sem0) if g % 2 == 0 else (pk1, sem1)
        pltpu.make_async_copy(packed.at[pl.ds(base, 16)], pk, sem).wait()

        def row(r, ofv):
            for l in range(8):
                pkt = pk[r, pl.ds(l * 16, 16)]
                keep = pkt >= 0
                pos = ofv + plsc.cumsum(keep.astype(jnp.int32)) - 1
                plsc.store_scatter(pend, [pos], pkt, mask=keep)
                ofv = ofv + plsc.all_reduce_population_count(keep)
            return ofv

        off = lax.fori_loop(0, 16, row, off)
        if g + 2 < 25:
            pltpu.async_copy(packed.at[pl.ds(base + (g + 2) * 16, 16)], pk,
                             sem)
    pend_end = (off + 1023) & ~1023
    lanes = jnp.arange(16, dtype=jnp.int32)
    neg1 = jnp.full((16,), -1, jnp.int32)
    for j in range(64):
        pos = off + lanes + j * 16
        plsc.store_scatter(pend, [pos], neg1, mask=pos < pend_end)
    nch = jnp.right_shift(pend_end, 10)
    for j in range(8):
        cbuf[pl.ds(j * 16, 16)] = nch
    pltpu.sync_copy(cbuf, ccnt_out.at[wid])
    pltpu.sync_copy(pend, cpk_out.at[wid])


_sc_comp = pl.kernel(
    _sc_comp_body,
    out_type=(jax.ShapeDtypeStruct((NW, 51200), jnp.int32),
              jax.ShapeDtypeStruct((NW, 128), jnp.int32)),
    mesh=_MESH,
    scratch_types=[
        pltpu.VMEM((51200,), jnp.int32),
        pltpu.VMEM((16, 128), jnp.int32),
        pltpu.VMEM((16, 128), jnp.int32),
        pltpu.VMEM((128,), jnp.int32),
        pltpu.SemaphoreType.DMA,
        pltpu.SemaphoreType.DMA,
    ],
    compiler_params=_SC_PARAMS,
)


def _sc_edge_body(cpk, ccnt, thT, users2d, zT_out, thuT_out,
                  thbuf, acc, pk0, pk1, pk2, pk3, ubuf, obuf, cntb,
                  sem0, sem1, sem2, sem3):
    c = lax.axis_index("c")
    s = lax.axis_index("s")
    wid = c * NS + s

    pltpu.sync_copy(thT.at[wid], thbuf)
    pltpu.sync_copy(ccnt, cntb)

    def zero(i, carry):
        acc[pl.ds(i * 16, 16)] = jnp.zeros((16,), jnp.float32)
        return carry

    lax.fori_loop(0, TP // 16, zero, 0)

    pks = (pk0, pk1, pk2, pk3)
    sems = (sem0, sem1, sem2, sem3)
    trips = [cntb[w, pl.ds(0, 16)][0] for w in range(NW)]

    def prime(w):
        pair = w % 2
        trip = trips[w]

        @pl.when(trip > 0)
        def _():
            pltpu.async_copy(cpk.at[w, pl.ds(0, 1024)], pks[2 * pair],
                             sems[2 * pair])

        @pl.when(trip > 1)
        def _():
            pltpu.async_copy(cpk.at[w, pl.ds(1024, 1024)], pks[2 * pair + 1],
                             sems[2 * pair + 1])

    prime(0)
    for w in range(NW):
        pair = w % 2
        if w + 1 < NW:
            prime(w + 1)
        trip = trips[w]

        def duo(i, carry):
            for h in (0, 1):
                pk = pks[2 * pair + h]
                sem = sems[2 * pair + h]
                g = 2 * i + h

                @pl.when(g < trip)
                def _():
                    pltpu.make_async_copy(cpk.at[w, pl.ds(0, 1024)], pk,
                                          sem).wait()

                    def row(v, c2):
                        for l in range(4):
                            pkt = pk[pl.ds(v * 64 + l * 16, 16)]
                            keep = pkt >= 0
                            rn = jnp.right_shift(pkt, 16)
                            cn = pkt & 0x7FFF
                            vv = plsc.load_gather(thbuf, [cn], mask=keep)
                            plsc.addupdate_scatter(acc, [rn], vv, mask=keep)
                        return c2

                    lax.fori_loop(0, 16, row, 0)

                    @pl.when(g + 2 < trip)
                    def _():
                        pltpu.async_copy(
                            cpk.at[w, pl.ds((g + 2) * 1024, 1024)], pk, sem)
            return carry

        lax.fori_loop(0, (trip + 1) // 2, duo, 0)
    pltpu.sync_copy(acc, zT_out.at[wid])

    def thu(ci, carry):
        pltpu.sync_copy(users2d.at[pl.ds(ci * 8, 8)], ubuf)
        for r8 in range(8):
            for l in range(8):
                idx = ubuf[r8, pl.ds(l * 16, 16)]
                obuf[pl.ds(r8 * 128 + l * 16, 16)] = plsc.load_gather(
                    thbuf, [idx])
        pltpu.sync_copy(obuf, thuT_out.at[wid, pl.ds(ci * 1024, 1024)])
        return carry

    lax.fori_loop(0, B // 1024, thu, 0)


_sc_edge = pl.kernel(
    _sc_edge_body,
    out_type=(jax.ShapeDtypeStruct((K, TP), jnp.float32),
              jax.ShapeDtypeStruct((K, B), jnp.float32)),
    mesh=_MESH,
    scratch_types=[
        pltpu.VMEM((TP,), jnp.float32),
        pltpu.VMEM((TP,), jnp.float32),
        pltpu.VMEM((1024,), jnp.int32),
        pltpu.VMEM((1024,), jnp.int32),
        pltpu.VMEM((1024,), jnp.int32),
        pltpu.VMEM((1024,), jnp.int32),
        pltpu.VMEM((8, 128), jnp.int32),
        pltpu.VMEM((1024,), jnp.float32),
        pltpu.VMEM((NW, 128), jnp.int32),
        pltpu.SemaphoreType.DMA,
        pltpu.SemaphoreType.DMA,
        pltpu.SemaphoreType.DMA,
        pltpu.SemaphoreType.DMA,
    ],
    compiler_params=_SC_PARAMS,
)


def _sc_emb_body(ueT3, ieT3, users2d, pos2d, neg2d, uT, pT, nT,
                 tb0, tb1, tb2, tb3, idxb, ob, sem0):
    c = lax.axis_index("c")
    s = lax.axis_index("s")
    wid = c * NS + s
    tbs = (tb0, tb1, tb2, tb3)

    for tbl3, jobs in ((ueT3, ((users2d, uT),)),
                       (ieT3, ((pos2d, pT), (neg2d, nT)))):
        cps = [pltpu.async_copy(tbl3.at[wid, cc], tbs[cc], sem0)
               for cc in range(4)]
        for cp in cps:
            cp.wait()
        for idx2d, out in jobs:
            def emb(ci, carry):
                pltpu.sync_copy(idx2d.at[pl.ds(ci * 8, 8)], idxb)
                for r8 in range(8):
                    for l in range(8):
                        sl = pl.ds(r8 * 128 + l * 16, 16)
                        idx = idxb[r8, pl.ds(l * 16, 16)]
                        for cc in range(4):
                            ob[cc, sl] = plsc.load_gather(tbs[cc], [idx])
                for cc in range(4):
                    pltpu.sync_copy(
                        ob.at[cc],
                        out.at[4 * wid + cc, pl.ds(ci * 1024, 1024)])
                return carry

            lax.fori_loop(0, B // 1024, emb, 0)


_sc_emb = pl.kernel(
    _sc_emb_body,
    out_type=(jax.ShapeDtypeStruct((EMB, B), jnp.float32),
              jax.ShapeDtypeStruct((EMB, B), jnp.float32),
              jax.ShapeDtypeStruct((EMB, B), jnp.float32)),
    mesh=_MESH,
    scratch_types=[
        pltpu.VMEM((TP,), jnp.float32),
        pltpu.VMEM((TP,), jnp.float32),
        pltpu.VMEM((TP,), jnp.float32),
        pltpu.VMEM((TP,), jnp.float32),
        pltpu.VMEM((8, 128), jnp.int32),
        pltpu.VMEM((4, 1024), jnp.float32),
        pltpu.SemaphoreType.DMA,
    ],
    compiler_params=_SC_PARAMS,
)


def _sc_z1_body(z1T, pos2d, neg2d, z1pT, z1nT, zrow, idxb, ob, sem0):
    c = lax.axis_index("c")
    s = lax.axis_index("s")
    wid = c * NS + s
    pltpu.sync_copy(z1T.at[wid], zrow)
    for idx2d, out in ((pos2d, z1pT), (neg2d, z1nT)):
        def gth(ci, carry):
            pltpu.sync_copy(idx2d.at[pl.ds(ci * 8, 8)], idxb)
            for r8 in range(8):
                for l in range(8):
                    idx = idxb[r8, pl.ds(l * 16, 16)]
                    ob[pl.ds(r8 * 128 + l * 16, 16)] = plsc.load_gather(
                        zrow, [idx])
            pltpu.sync_copy(ob, out.at[wid, pl.ds(ci * 1024, 1024)])
            return carry

        lax.fori_loop(0, B // 1024, gth, 0)


_sc_z1 = pl.kernel(
    _sc_z1_body,
    out_type=(jax.ShapeDtypeStruct((K, B), jnp.float32),
              jax.ShapeDtypeStruct((K, B), jnp.float32)),
    mesh=_MESH,
    scratch_types=[
        pltpu.VMEM((TP,), jnp.float32),
        pltpu.VMEM((8, 128), jnp.int32),
        pltpu.VMEM((1024,), jnp.float32),
        pltpu.SemaphoreType.DMA,
    ],
    compiler_params=_SC_PARAMS,
)


def _tc2a_body(z_ref, w1_ref, w2_ref, sc_ref, out_ref):
    t = z_ref[...] * sc_ref[0, 0] * w1_ref[...] + w2_ref[...]
    out_ref[...] = 1.0 / (1.0 + jnp.exp(-t))


_tc2a = pl.pallas_call(
    _tc2a_body,
    out_shape=jax.ShapeDtypeStruct((K, TP), jnp.float32),
)


def _tc2b_body(u_ref, p_ref, n_ref, ps_ref, ns_ref, sq_ref):
    i = pl.program_id(0)
    u = u_ref[...]
    p = p_ref[...]
    n = n_ref[...]
    ps_ref[...] = jnp.sum(u * p, axis=0, keepdims=True)
    ns_ref[...] = jnp.sum(u * n, axis=0, keepdims=True)
    acc = jnp.sum(u * u) + jnp.sum(p * p) + jnp.sum(n * n)

    @pl.when(i == 0)
    def _():
        sq_ref[...] = acc.reshape(1, 1)

    @pl.when(i > 0)
    def _():
        sq_ref[...] += acc.reshape(1, 1)


_tc2b = pl.pallas_call(
    _tc2b_body,
    grid=(8,),
    in_specs=[pl.BlockSpec((EMB, B // 8), lambda i: (0, i))] * 3,
    out_specs=(pl.BlockSpec((1, B // 8), lambda i: (0, i)),
               pl.BlockSpec((1, B // 8), lambda i: (0, i)),
               pl.BlockSpec((1, 1), lambda i: (0, 0))),
    out_shape=(jax.ShapeDtypeStruct((1, B), jnp.float32),
               jax.ShapeDtypeStruct((1, B), jnp.float32),
               jax.ShapeDtypeStruct((1, 1), jnp.float32)),
)


def _tc3_body(ps_ref, ns_ref, thu_ref, z1p_ref, z1n_ref, s2_ref, s3_ref,
              sq_ref, out_ref, acc):
    i = pl.program_id(0)
    thu = thu_ref[...]
    gp = jnp.sum(thu * z1p_ref[...], axis=0, keepdims=True)
    gn = jnp.sum(thu * z1n_ref[...], axis=0, keepdims=True)
    rp = 1.0 / (1.0 + jnp.exp(-ps_ref[...]))
    rn = 1.0 / (1.0 + jnp.exp(-ns_ref[...]))
    mf = jnp.sum(gp * -jnp.log(rp)) + jnp.sum(gn * -jnp.log(1.0 - rn))
    l1 = -float(np.log(np.float32(0.001)))
    l0 = -float(np.log(np.float32(1.0) - np.float32(0.001)))
    unk = l1 * jnp.sum(1.0 - gp) + l0 * jnp.sum(1.0 - gn)
    gu = -(jnp.sum(gp * jnp.log(gp) + (1.0 - gp) * jnp.log(1.0 - gp))
           + jnp.sum(gn * jnp.log(gn) + (1.0 - gn) * jnp.log(1.0 - gn)))

    @pl.when(i == 0)
    def _():
        acc[0] = mf
        acc[1] = unk
        acc[2] = gu

    @pl.when(i > 0)
    def _():
        acc[0] += mf
        acc[1] += unk
        acc[2] += gu

    @pl.when(i == 7)
    def _():
        rl1 = 0.5 * sq_ref[0, 0] / float(B)
        rl2 = 0.5 * s2_ref[0, 0] / float(NI)
        rl3 = 0.5 * s3_ref[0, 0] / float(NU)
        reg = WD * (rl1 + rl3) + 0.1 * rl2
        inv = 1.0 / float(2 * B)
        out_ref[...] = (acc[0] * inv + 0.1 * (acc[1] * inv - acc[2] * inv)
                        + reg).reshape(1, 1)


_tc3 = pl.pallas_call(
    _tc3_body,
    grid=(8,),
    in_specs=[pl.BlockSpec((1, B // 8), lambda i: (0, i)),
              pl.BlockSpec((1, B // 8), lambda i: (0, i)),
              pl.BlockSpec((K, B // 8), lambda i: (0, i)),
              pl.BlockSpec((K, B // 8), lambda i: (0, i)),
              pl.BlockSpec((K, B // 8), lambda i: (0, i)),
              pl.BlockSpec((1, 1), lambda i: (0, 0)),
              pl.BlockSpec((1, 1), lambda i: (0, 0)),
              pl.BlockSpec((1, 1), lambda i: (0, 0))],
    out_specs=pl.BlockSpec((1, 1), lambda i: (0, 0)),
    out_shape=jax.ShapeDtypeStruct((1, 1), jnp.float32),
    scratch_shapes=[pltpu.SMEM((4,), jnp.float32)],
)


def kernel(users, positive_items, negative_items, edge_index, edge_values,
           user_embedding, item_embedding, theta_user, w1, w2):
    users = users.astype(jnp.int32)
    positive_items = positive_items.astype(jnp.int32)
    negative_items = negative_items.astype(jnp.int32)
    edge_index = edge_index.astype(jnp.int32)

    pad = EP - E
    rows2d = jnp.concatenate(
        [edge_index[0], jnp.zeros((pad,), dtype=jnp.int32)]).reshape(CR, 128)
    cols2d = jnp.concatenate(
        [edge_index[1], jnp.zeros((pad,), dtype=jnp.int32)]).reshape(CR, 128)
    packed = _tc_pack(rows2d, cols2d)
    cpk, ccnt = _sc_comp(packed)

    zpad = jnp.zeros((TP - NU, EMB), jnp.float32)
    uep = jnp.concatenate([user_embedding, zpad])
    iep = jnp.concatenate([item_embedding, zpad])
    tup = jnp.concatenate([theta_user, jnp.zeros((TP - NU, K), jnp.float32)])
    ueT, ieT, thT, s3, s2 = _tc_prep(uep, iep, tup,
                                     w1.reshape(8, NU // 8),
                                     w2.reshape(8, NI // 8))

    users2d = users.reshape(BCR, 128)
    pos2d = positive_items.reshape(BCR, 128)
    neg2d = negative_items.reshape(BCR, 128)

    zT, thuT = _sc_edge(cpk, ccnt, thT, users2d)
    uT, pT, nT = _sc_emb(ueT.reshape(NW, 4, TP), ieT.reshape(NW, 4, TP),
                         users2d, pos2d, neg2d)

    scale = edge_values[0].reshape(1, 1)
    w1p = jnp.pad(w1.reshape(1, NU), ((0, 0), (0, TP - NU)))
    w2p = jnp.pad(w2.reshape(1, NI), ((0, 0), (0, TP - NU)))
    z1T = _tc2a(zT, w1p, w2p, scale)
    z1pT, z1nT = _sc_z1(z1T, pos2d, neg2d)

    ps, ns, sq = _tc2b(uT, pT, nT)
    loss = _tc3(ps, ns, thuT, z1pT, z1nT, s2, s3, sq)
    return loss.reshape(())
